# baseline (speedup 1.0000x reference)
import jax
import jax.numpy as jnp
from jax.experimental import pallas as pl
from jax.experimental.pallas import tpu as pltpu

N = 10000
C = 2000
FEA = 128
TGT = 64


def _simple_fwd(p, x):
    for W, b in p["layers"]:
        x = jax.nn.leaky_relu(x @ W + b, negative_slope=0.01)
    W, b = p["out"]
    return x @ W + b


def _wattn(p, fea, index, weights, nseg):
    gate = _simple_fwd(p["gate"], fea)
    gmax = jax.ops.segment_max(gate, index, num_segments=nseg)
    gate = gate - gmax[index]
    gate = weights * jnp.exp(gate)
    den = jax.ops.segment_sum(gate, index, num_segments=nseg)
    gate = gate / (den[index] + 1e-13)
    msg = _simple_fwd(p["msg"], fea)
    return jax.ops.segment_sum(gate * msg, index, num_segments=nseg)


def _res_mlp_kernel(x_ref, *refs):
    out_ref = refs[-1]
    ws = refs[:-1]
    x = x_ref[...]
    i = 0
    for li, has_res in enumerate((True, True, False, True)):
        W = ws[i][...]
        b = ws[i + 1][...]
        i += 2
        if has_res:
            Wres = ws[i][...]
            i += 1
            res = jnp.dot(x, Wres, preferred_element_type=jnp.float32)
        else:
            res = x
        x = jax.nn.relu(jnp.dot(x, W, preferred_element_type=jnp.float32) + b) + res
    Wo = ws[i][...]
    bo = ws[i + 1][...]
    out_ref[...] = jnp.dot(x, Wo, preferred_element_type=jnp.float32) + bo


def _res_mlp(x, res_layers, out_wb):
    CP = 2048
    xp = jnp.zeros((CP, x.shape[1]), jnp.float32).at[: x.shape[0]].set(x)
    args = []
    for (W, b, Wres) in res_layers:
        args.append(W)
        args.append(b.reshape(1, -1))
        if Wres is not None:
            args.append(Wres)
    Wo, bo = out_wb
    args.append(Wo)
    args.append(bo.reshape(1, -1))
    out = pl.pallas_call(
        _res_mlp_kernel,
        out_shape=jax.ShapeDtypeStruct((CP, TGT), jnp.float32),
    )(xp, *args)
    return out[: x.shape[0]]


def kernel(prec_weights, orig_prec_fea, self_fea_idx, nbr_fea_idx, reaction_prec_idx, actions, params):
    fea = orig_prec_fea @ params["embed"]
    fea = jnp.concatenate([fea, prec_weights], axis=1)
    for heads in params["graphs"]:
        nbr_w = prec_weights[nbr_fea_idx]
        nbr_f = fea[nbr_fea_idx]
        self_f = fea[self_fea_idx]
        glob = actions[reaction_prec_idx[self_fea_idx]]
        edge = jnp.concatenate([self_f, nbr_f, glob], axis=1)
        head_out = [_wattn(h, edge, self_fea_idx, nbr_w, N) for h in heads]
        fea = jnp.mean(jnp.stack(head_out), axis=0) + fea
    head_out = [_wattn(h, fea, reaction_prec_idx, prec_weights, C) for h in params["cry"]]
    x = jnp.mean(jnp.stack(head_out), axis=0)
    return _res_mlp(x, params["res"], params["out"])


# speedup vs baseline: 2.3580x; 2.3580x over previous
import functools

import jax
import jax.numpy as jnp
from jax.experimental import pallas as pl
from jax.experimental.pallas import tpu as pltpu

N = 10000
M = 160000
C = 2000
ORIG = 200
FEA = 128
LAT = 32
HEADS = 3
TGT = 64
HID = 256

BM = 640
BN = 400
NEG = -3.0e38


def _leaky(x):
    return jnp.where(x > 0, x, 0.01 * x)


def _embed_body(orig_ref, pw_ref, rxn_ref, Wemb_ref, act_ref, fea_ref, glob_ref, tself_ref):
    orig = orig_ref[...]
    emb = jnp.dot(orig, Wemb_ref[...], preferred_element_type=jnp.float32)
    col = jax.lax.broadcasted_iota(jnp.int32, (BN, FEA), 1)
    fea = jnp.where(col == FEA - 1, pw_ref[...], emb)
    fea_ref[...] = fea
    rxn = rxn_ref[0, 0, :]
    z = (rxn[:, None] == jax.lax.broadcasted_iota(jnp.int32, (BN, C), 1)).astype(jnp.float32)
    glob = jax.lax.dot_general(z, act_ref[...], (((1,), (0,)), ((), ())),
                               preferred_element_type=jnp.float32)
    glob_ref[...] = glob
    tself_ref[...] = jnp.concatenate([fea, glob], axis=1)


def _embed(orig, pw, rxn, Wemb, actions):
    Wpad = jnp.concatenate([Wemb, jnp.zeros((ORIG, 1), jnp.float32)], axis=1)
    rxn3 = rxn.reshape(N // BN, 1, BN)
    grid = (N // BN,)
    return pl.pallas_call(
        _embed_body,
        grid=grid,
        in_specs=[
            pl.BlockSpec((BN, ORIG), lambda i: (i, 0)),
            pl.BlockSpec((BN, 1), lambda i: (i, 0)),
            pl.BlockSpec((1, 1, BN), lambda i: (i, 0, 0)),
            pl.BlockSpec((ORIG, FEA), lambda i: (0, 0)),
            pl.BlockSpec((C, LAT), lambda i: (0, 0)),
        ],
        out_specs=[
            pl.BlockSpec((BN, FEA), lambda i: (i, 0)),
            pl.BlockSpec((BN, LAT), lambda i: (i, 0)),
            pl.BlockSpec((BN, FEA + LAT), lambda i: (i, 0)),
        ],
        out_shape=[
            jax.ShapeDtypeStruct((N, FEA), jnp.float32),
            jax.ShapeDtypeStruct((N, LAT), jnp.float32),
            jax.ShapeDtypeStruct((N, FEA + LAT), jnp.float32),
        ],
    )(orig, pw, rxn3, Wpad, actions)


def _gates_body(es_ref, en_ref, Ws_ref, Wn_ref, b1_ref, W2_ref, b2_ref, out_ref):
    h = jnp.dot(es_ref[...], Ws_ref[...], preferred_element_type=jnp.float32)
    h += jnp.dot(en_ref[...], Wn_ref[...], preferred_element_type=jnp.float32)
    h = _leaky(h + b1_ref[...])
    g = jnp.dot(h, W2_ref[...], preferred_element_type=jnp.float32) + b2_ref[...]
    out_ref[...] = g.T


def _gates(eself, enbr, Ws, Wn, b1, W2, b2):
    grid = (M // BM,)
    return pl.pallas_call(
        _gates_body,
        grid=grid,
        in_specs=[
            pl.BlockSpec((BM, FEA + LAT), lambda i: (i, 0)),
            pl.BlockSpec((BM, FEA), lambda i: (i, 0)),
            pl.BlockSpec((FEA + LAT, HEADS * HID), lambda i: (0, 0)),
            pl.BlockSpec((FEA, HEADS * HID), lambda i: (0, 0)),
            pl.BlockSpec((1, HEADS * HID), lambda i: (0, 0)),
            pl.BlockSpec((HEADS * HID, HEADS), lambda i: (0, 0)),
            pl.BlockSpec((1, HEADS), lambda i: (0, 0)),
        ],
        out_specs=pl.BlockSpec((HEADS, BM), lambda i: (0, i)),
        out_shape=jax.ShapeDtypeStruct((HEADS, M), jnp.float32),
    )(eself, enbr, Ws, Wn, b1, W2, b2)


def _msgs_body(es_ref, en_ref, w_ref, Ws_ref, Wn_ref, b1_ref, W2_ref, b2_ref, out_ref):
    h = jnp.dot(es_ref[...], Ws_ref[...], preferred_element_type=jnp.float32)
    h += jnp.dot(en_ref[...], Wn_ref[...], preferred_element_type=jnp.float32)
    h = _leaky(h + b1_ref[...])
    for hh in range(HEADS):
        m = jnp.dot(h[:, hh * HID:(hh + 1) * HID], W2_ref[hh],
                    preferred_element_type=jnp.float32) + b2_ref[hh]
        out_ref[hh] = m * w_ref[hh][:, None]


def _msgs(eself, enbr, w, Ws, Wn, b1, W2, b2):
    grid = (M // BM,)
    return pl.pallas_call(
        _msgs_body,
        grid=grid,
        in_specs=[
            pl.BlockSpec((BM, FEA + LAT), lambda i: (i, 0)),
            pl.BlockSpec((BM, FEA), lambda i: (i, 0)),
            pl.BlockSpec((HEADS, BM), lambda i: (0, i)),
            pl.BlockSpec((FEA + LAT, HEADS * HID), lambda i: (0, 0)),
            pl.BlockSpec((FEA, HEADS * HID), lambda i: (0, 0)),
            pl.BlockSpec((1, HEADS * HID), lambda i: (0, 0)),
            pl.BlockSpec((HEADS, HID, FEA), lambda i: (0, 0, 0)),
            pl.BlockSpec((HEADS, 1, FEA), lambda i: (0, 0, 0)),
        ],
        out_specs=pl.BlockSpec((HEADS, BM, FEA), lambda i: (0, i, 0)),
        out_shape=jax.ShapeDtypeStruct((HEADS, M, FEA), jnp.float32),
    )(eself, enbr, w, Ws, Wn, b1, W2, b2)


def _norm_body(num_ref, den_ref, fea_ref, glob_ref, fea_o, tself_o):
    num = num_ref[0] + num_ref[1]
    den = den_ref[0] + den_ref[1]
    acc = jnp.zeros((BN, FEA), jnp.float32)
    for hh in range(HEADS):
        acc += num[hh] / (den[hh] + 1e-13)
    fea = acc * (1.0 / HEADS) + fea_ref[...]
    fea_o[...] = fea
    tself_o[...] = jnp.concatenate([fea, glob_ref[...]], axis=1)


def _norm(num, den, fea, glob):
    grid = (N // BN,)
    return pl.pallas_call(
        _norm_body,
        grid=grid,
        in_specs=[
            pl.BlockSpec((2, HEADS, BN, FEA), lambda i: (0, 0, i, 0)),
            pl.BlockSpec((2, HEADS, BN, 1), lambda i: (0, 0, i, 0)),
            pl.BlockSpec((BN, FEA), lambda i: (i, 0)),
            pl.BlockSpec((BN, LAT), lambda i: (i, 0)),
        ],
        out_specs=[
            pl.BlockSpec((BN, FEA), lambda i: (i, 0)),
            pl.BlockSpec((BN, FEA + LAT), lambda i: (i, 0)),
        ],
        out_shape=[
            jax.ShapeDtypeStruct((N, FEA), jnp.float32),
            jax.ShapeDtypeStruct((N, FEA + LAT), jnp.float32),
        ],
    )(num, den, fea, glob)


def _cry_a_body(fea_ref, rxn_ref, Ws_ref, b1_ref, W2_ref, b2_ref,
                gates_o, gmax_o, gmax_acc):
    i = pl.program_id(0)

    @pl.when(i == 0)
    def _():
        gmax_acc[...] = jnp.full((HEADS, C), NEG, jnp.float32)

    h = _leaky(jnp.dot(fea_ref[...], Ws_ref[...], preferred_element_type=jnp.float32)
               + b1_ref[...])
    g = jnp.dot(h, W2_ref[...], preferred_element_type=jnp.float32) + b2_ref[...]
    gates_o[...] = g
    rxn = rxn_ref[0, 0, :]
    z = rxn[:, None] == jax.lax.broadcasted_iota(jnp.int32, (BN, C), 1)
    for hh in range(HEADS):
        masked = jnp.where(z, g[:, hh:hh + 1], NEG)
        gmax_acc[hh, :] = jnp.maximum(gmax_acc[hh, :], jnp.max(masked, axis=0))

    @pl.when(i == pl.num_programs(0) - 1)
    def _():
        gmax_o[...] = gmax_acc[...]


def _cry_b_body(fea_ref, pw_ref, rxn_ref, gates_ref, gmax_ref,
                Ws_ref, b1_ref, W2_ref, b2_ref, x_o, acc):
    i = pl.program_id(0)

    @pl.when(i == 0)
    def _():
        acc[...] = jnp.zeros((HEADS, C, FEA + 16), jnp.float32)

    h = _leaky(jnp.dot(fea_ref[...], Ws_ref[...], preferred_element_type=jnp.float32)
               + b1_ref[...])
    rxn = rxn_ref[0, 0, :]
    z = rxn[:, None] == jax.lax.broadcasted_iota(jnp.int32, (BN, C), 1)
    zf = z.astype(jnp.float32)
    pw = pw_ref[...]
    for hh in range(HEADS):
        gm = jnp.max(jnp.where(z, gmax_ref[hh, :][None, :], NEG), axis=1)
        e = pw[:, 0] * jnp.exp(gates_ref[:, hh] - gm)
        m = jnp.dot(h[:, hh * HID:(hh + 1) * HID], W2_ref[hh],
                    preferred_element_type=jnp.float32) + b2_ref[hh]
        val = jnp.concatenate(
            [m * e[:, None], e[:, None], jnp.zeros((BN, 15), jnp.float32)], axis=1)
        acc[hh] += jax.lax.dot_general(zf, val, (((0,), (0,)), ((), ())),
                                       preferred_element_type=jnp.float32)

    @pl.when(i == pl.num_programs(0) - 1)
    def _():
        x = jnp.zeros((C, FEA), jnp.float32)
        for hh in range(HEADS):
            a = acc[hh]
            x += a[:, :FEA] / (a[:, FEA:FEA + 1] + 1e-13)
        x_o[...] = x * (1.0 / HEADS)


def _cry(fea, pw, rxn, cry_params):
    Ws = jnp.concatenate([cry_params[h]["gate"]["layers"][0][0] for h in range(HEADS)], axis=1)
    b1 = jnp.concatenate([cry_params[h]["gate"]["layers"][0][1] for h in range(HEADS)])
    W2 = jnp.zeros((HEADS * HID, HEADS), jnp.float32)
    for h in range(HEADS):
        W2 = W2.at[h * HID:(h + 1) * HID, h].set(cry_params[h]["gate"]["out"][0][:, 0])
    b2 = jnp.stack([cry_params[h]["gate"]["out"][1][0] for h in range(HEADS)])
    Wsm = jnp.concatenate([cry_params[h]["msg"]["layers"][0][0] for h in range(HEADS)], axis=1)
    b1m = jnp.concatenate([cry_params[h]["msg"]["layers"][0][1] for h in range(HEADS)])
    W2m = jnp.stack([cry_params[h]["msg"]["out"][0] for h in range(HEADS)])
    b2m = jnp.stack([cry_params[h]["msg"]["out"][1].reshape(1, FEA) for h in range(HEADS)])

    rxn3 = rxn.reshape(N // BN, 1, BN)
    grid = (N // BN,)
    gates, gmax = pl.pallas_call(
        _cry_a_body,
        grid=grid,
        in_specs=[
            pl.BlockSpec((BN, FEA), lambda i: (i, 0)),
            pl.BlockSpec((1, 1, BN), lambda i: (i, 0, 0)),
            pl.BlockSpec((FEA, HEADS * HID), lambda i: (0, 0)),
            pl.BlockSpec((1, HEADS * HID), lambda i: (0, 0)),
            pl.BlockSpec((HEADS * HID, HEADS), lambda i: (0, 0)),
            pl.BlockSpec((1, HEADS), lambda i: (0, 0)),
        ],
        out_specs=[
            pl.BlockSpec((BN, HEADS), lambda i: (i, 0)),
            pl.BlockSpec((HEADS, C), lambda i: (0, 0)),
        ],
        out_shape=[
            jax.ShapeDtypeStruct((N, HEADS), jnp.float32),
            jax.ShapeDtypeStruct((HEADS, C), jnp.float32),
        ],
        scratch_shapes=[pltpu.VMEM((HEADS, C), jnp.float32)],
    )(fea, rxn3, Ws, b1.reshape(1, -1), W2, b2.reshape(1, -1))

    x = pl.pallas_call(
        _cry_b_body,
        grid=grid,
        in_specs=[
            pl.BlockSpec((BN, FEA), lambda i: (i, 0)),
            pl.BlockSpec((BN, 1), lambda i: (i, 0)),
            pl.BlockSpec((1, 1, BN), lambda i: (i, 0, 0)),
            pl.BlockSpec((BN, HEADS), lambda i: (i, 0)),
            pl.BlockSpec((HEADS, C), lambda i: (0, 0)),
            pl.BlockSpec((FEA, HEADS * HID), lambda i: (0, 0)),
            pl.BlockSpec((1, HEADS * HID), lambda i: (0, 0)),
            pl.BlockSpec((HEADS, HID, FEA), lambda i: (0, 0, 0)),
            pl.BlockSpec((HEADS, 1, FEA), lambda i: (0, 0, 0)),
        ],
        out_specs=pl.BlockSpec((C, FEA), lambda i: (0, 0)),
        out_shape=jax.ShapeDtypeStruct((C, FEA), jnp.float32),
        scratch_shapes=[pltpu.VMEM((HEADS, C, FEA + 16), jnp.float32)],
    )(fea, pw, rxn3, gates, gmax, Wsm, b1m.reshape(1, -1), W2m, b2m)
    return x


def _res_mlp_kernel(x_ref, *refs):
    out_ref = refs[-1]
    ws = refs[:-1]
    x = x_ref[...]
    i = 0
    for li, has_res in enumerate((True, True, False, True)):
        W = ws[i][...]
        b = ws[i + 1][...]
        i += 2
        if has_res:
            Wres = ws[i][...]
            i += 1
            res = jnp.dot(x, Wres, preferred_element_type=jnp.float32)
        else:
            res = x
        x = jax.nn.relu(jnp.dot(x, W, preferred_element_type=jnp.float32) + b) + res
    Wo = ws[i][...]
    bo = ws[i + 1][...]
    out_ref[...] = jnp.dot(x, Wo, preferred_element_type=jnp.float32) + bo


def _res_mlp(x, res_layers, out_wb):
    CP = 2048
    xp = jnp.zeros((CP, x.shape[1]), jnp.float32).at[: x.shape[0]].set(x)
    args = []
    for (W, b, Wres) in res_layers:
        args.append(W)
        args.append(b.reshape(1, -1))
        if Wres is not None:
            args.append(Wres)
    Wo, bo = out_wb
    args.append(Wo)
    args.append(bo.reshape(1, -1))
    out = pl.pallas_call(
        _res_mlp_kernel,
        out_shape=jax.ShapeDtypeStruct((CP, TGT), jnp.float32),
    )(xp, *args)
    return out[: x.shape[0]]


def _pack_layer(heads):
    def cat(key, part):
        return jnp.concatenate([heads[h][key]["layers"][0][0][part] for h in range(HEADS)], axis=1)
    sl_self = slice(0, FEA)
    sl_nbr = slice(FEA, 2 * FEA)
    sl_glob = slice(2 * FEA, 2 * FEA + LAT)
    Wg_s = jnp.concatenate([cat("gate", sl_self), cat("gate", sl_glob)], axis=0)
    Wg_n = cat("gate", sl_nbr)
    b1g = jnp.concatenate([heads[h]["gate"]["layers"][0][1] for h in range(HEADS)])
    W2g = jnp.zeros((HEADS * HID, HEADS), jnp.float32)
    for h in range(HEADS):
        W2g = W2g.at[h * HID:(h + 1) * HID, h].set(heads[h]["gate"]["out"][0][:, 0])
    b2g = jnp.stack([heads[h]["gate"]["out"][1][0] for h in range(HEADS)])
    Wm_s = jnp.concatenate([cat("msg", sl_self), cat("msg", sl_glob)], axis=0)
    Wm_n = cat("msg", sl_nbr)
    b1m = jnp.concatenate([heads[h]["msg"]["layers"][0][1] for h in range(HEADS)])
    W2m = jnp.stack([heads[h]["msg"]["out"][0] for h in range(HEADS)])
    b2m = jnp.stack([heads[h]["msg"]["out"][1].reshape(1, FEA) for h in range(HEADS)])
    return (Wg_s, Wg_n, b1g.reshape(1, -1), W2g, b2g.reshape(1, -1),
            Wm_s, Wm_n, b1m.reshape(1, -1), W2m, b2m)


def kernel(prec_weights, orig_prec_fea, self_fea_idx, nbr_fea_idx, reaction_prec_idx, actions, params):
    pw = prec_weights
    rxn = reaction_prec_idx
    fea, glob, tself = _embed(orig_prec_fea, pw, rxn, params["embed"], actions)

    nbr_w = pw[nbr_fea_idx, 0]
    for heads in params["graphs"]:
        (Wg_s, Wg_n, b1g, W2g, b2g, Wm_s, Wm_n, b1m, W2m, b2m) = _pack_layer(heads)
        eself = tself[self_fea_idx]
        enbr = fea[nbr_fea_idx]
        gates = _gates(eself, enbr, Wg_s, Wg_n, b1g, W2g, b2g)
        gmax = jax.ops.segment_max(gates.T, self_fea_idx, num_segments=N)
        w = nbr_w[None, :] * jnp.exp(gates - gmax.T[:, self_fea_idx])
        wmsg = _msgs(eself, enbr, w, Wm_s, Wm_n, b1m, W2m, b2m)
        num = jax.ops.segment_sum(
            wmsg.transpose(1, 0, 2).reshape(M, HEADS * FEA), self_fea_idx,
            num_segments=N).reshape(N, HEADS, FEA).transpose(1, 0, 2)
        den = jax.ops.segment_sum(w.T, self_fea_idx, num_segments=N).T
        num2 = jnp.stack([num, jnp.zeros_like(num)])
        den2 = jnp.stack([den[..., None], jnp.zeros_like(den[..., None])])
        fea, tself = _norm(num2, den2, fea, glob)

    x = _cry(fea, pw, rxn, params["cry"])
    return _res_mlp(x, params["res"], params["out"])


# speedup vs baseline: 6.4134x; 2.7198x over previous
import dataclasses
import functools

import jax
import jax.numpy as jnp
from jax.experimental import pallas as pl
from jax.experimental.pallas import tpu as pltpu
from jax.experimental.pallas import tpu_sc as plsc

N = 10000
M = 160000
C = 2000
ORIG = 200
FEA = 128
LAT = 32
HEADS = 3
TGT = 64
HID = 256

BM = 640
BN = 400
NEG = -3.0e38


def _leaky(x):
    return jnp.where(x > 0, x, 0.01 * x)


def _embed_body(orig_ref, pw_ref, rxn_ref, Wemb_ref, act_ref, fea_ref, glob_ref, tself_ref):
    orig = orig_ref[...]
    emb = jnp.dot(orig, Wemb_ref[...], preferred_element_type=jnp.float32)
    col = jax.lax.broadcasted_iota(jnp.int32, (BN, FEA), 1)
    fea = jnp.where(col == FEA - 1, pw_ref[...], emb)
    fea_ref[...] = fea
    rxn = rxn_ref[0, 0, :]
    z = (rxn[:, None] == jax.lax.broadcasted_iota(jnp.int32, (BN, C), 1)).astype(jnp.float32)
    glob = jax.lax.dot_general(z, act_ref[...], (((1,), (0,)), ((), ())),
                               preferred_element_type=jnp.float32)
    glob_ref[...] = glob
    tself_ref[...] = jnp.concatenate(
        [fea, glob, jnp.zeros((BN, 96), jnp.float32)], axis=1)


def _embed(orig, pw, rxn, Wemb, actions):
    Wpad = jnp.concatenate([Wemb, jnp.zeros((ORIG, 1), jnp.float32)], axis=1)
    rxn3 = rxn.reshape(N // BN, 1, BN)
    grid = (N // BN,)
    return pl.pallas_call(
        _embed_body,
        grid=grid,
        in_specs=[
            pl.BlockSpec((BN, ORIG), lambda i: (i, 0)),
            pl.BlockSpec((BN, 1), lambda i: (i, 0)),
            pl.BlockSpec((1, 1, BN), lambda i: (i, 0, 0)),
            pl.BlockSpec((ORIG, FEA), lambda i: (0, 0)),
            pl.BlockSpec((C, LAT), lambda i: (0, 0)),
        ],
        out_specs=[
            pl.BlockSpec((BN, FEA), lambda i: (i, 0)),
            pl.BlockSpec((BN, LAT), lambda i: (i, 0)),
            pl.BlockSpec((BN, 256), lambda i: (i, 0)),
        ],
        out_shape=[
            jax.ShapeDtypeStruct((N, FEA), jnp.float32),
            jax.ShapeDtypeStruct((N, LAT), jnp.float32),
            jax.ShapeDtypeStruct((N, 256), jnp.float32),
        ],
    )(orig, pw, rxn3, Wpad, actions)


def _gates_body(es_ref, en_ref, Ws_ref, Wn_ref, b1_ref, W2_ref, b2_ref, out_ref):
    h = jnp.dot(es_ref[:, :160], Ws_ref[...], preferred_element_type=jnp.float32)
    h += jnp.dot(en_ref[...], Wn_ref[...], preferred_element_type=jnp.float32)
    h = _leaky(h + b1_ref[...])
    g = jnp.dot(h, W2_ref[...], preferred_element_type=jnp.float32) + b2_ref[...]
    out_ref[...] = g.T


def _gates(es, en, Ws, Wn, b1, W2, b2):
    grid = (M // BM,)
    return pl.pallas_call(
        _gates_body,
        grid=grid,
        in_specs=[
            pl.BlockSpec((BM, 256), lambda i: (i, 0)),
            pl.BlockSpec((BM, FEA), lambda i: (i, 0)),
            pl.BlockSpec((160, HEADS * HID), lambda i: (0, 0)),
            pl.BlockSpec((FEA, HEADS * HID), lambda i: (0, 0)),
            pl.BlockSpec((1, HEADS * HID), lambda i: (0, 0)),
            pl.BlockSpec((HEADS * HID, HEADS), lambda i: (0, 0)),
            pl.BlockSpec((1, HEADS), lambda i: (0, 0)),
        ],
        out_specs=pl.BlockSpec((HEADS, BM), lambda i: (0, i)),
        out_shape=jax.ShapeDtypeStruct((HEADS, M), jnp.float32),
    )(es, en, Ws, Wn, b1, W2, b2)


def _msgs_body(es_ref, en_ref, w_ref, Ws_ref, Wn_ref, b1_ref, W2_ref, b2_ref, out_ref, den_ref):
    h = jnp.dot(es_ref[:, :160], Ws_ref[...], preferred_element_type=jnp.float32)
    h += jnp.dot(en_ref[...], Wn_ref[...], preferred_element_type=jnp.float32)
    h = _leaky(h + b1_ref[...])
    dens = []
    for hh in range(HEADS):
        m = jnp.dot(h[:, hh * HID:(hh + 1) * HID], W2_ref[hh],
                    preferred_element_type=jnp.float32) + b2_ref[hh]
        wv = w_ref[hh][:, None]
        out_ref[hh] = m * wv
        dens.append(jnp.broadcast_to(wv, (BM, 16)))
    den_ref[...] = jnp.concatenate(
        dens + [jnp.zeros((BM, 128 - 16 * HEADS), jnp.float32)], axis=1)


def _msgs(es, en, w, Ws, Wn, b1, W2, b2):
    grid = (M // BM,)
    return pl.pallas_call(
        _msgs_body,
        grid=grid,
        in_specs=[
            pl.BlockSpec((BM, 256), lambda i: (i, 0)),
            pl.BlockSpec((BM, FEA), lambda i: (i, 0)),
            pl.BlockSpec((HEADS, BM), lambda i: (0, i)),
            pl.BlockSpec((160, HEADS * HID), lambda i: (0, 0)),
            pl.BlockSpec((FEA, HEADS * HID), lambda i: (0, 0)),
            pl.BlockSpec((1, HEADS * HID), lambda i: (0, 0)),
            pl.BlockSpec((HEADS, HID, FEA), lambda i: (0, 0, 0)),
            pl.BlockSpec((HEADS, 1, FEA), lambda i: (0, 0, 0)),
        ],
        out_specs=[
            pl.BlockSpec((HEADS, BM, FEA), lambda i: (0, i, 0)),
            pl.BlockSpec((BM, FEA), lambda i: (i, 0)),
        ],
        out_shape=[
            jax.ShapeDtypeStruct((HEADS, M, FEA), jnp.float32),
            jax.ShapeDtypeStruct((M, FEA), jnp.float32),
        ],
    )(es, en, w, Ws, Wn, b1, W2, b2)


def _norm_body(num_ref, den_ref, fea_ref, glob_ref, fea_o, tself_o):
    av = num_ref[0] + num_ref[1]
    denp = den_ref[0] + den_ref[1]
    acc = jnp.zeros((BN, FEA), jnp.float32)
    for hh in range(HEADS):
        acc += av[hh] / (denp[:, 16 * hh:16 * hh + 1] + 1e-13)
    fea = acc * (1.0 / HEADS) + fea_ref[...]
    fea_o[...] = fea
    tself_o[...] = jnp.concatenate(
        [fea, glob_ref[...], jnp.zeros((BN, 96), jnp.float32)], axis=1)


def _norm(onum, oden, fea, glob):
    grid = (N // BN,)
    return pl.pallas_call(
        _norm_body,
        grid=grid,
        in_specs=[
            pl.BlockSpec((2, HEADS, BN, FEA), lambda i: (0, 0, i, 0)),
            pl.BlockSpec((2, BN, FEA), lambda i: (0, i, 0)),
            pl.BlockSpec((BN, FEA), lambda i: (i, 0)),
            pl.BlockSpec((BN, LAT), lambda i: (i, 0)),
        ],
        out_specs=[
            pl.BlockSpec((BN, FEA), lambda i: (i, 0)),
            pl.BlockSpec((BN, 256), lambda i: (i, 0)),
        ],
        out_shape=[
            jax.ShapeDtypeStruct((N, FEA), jnp.float32),
            jax.ShapeDtypeStruct((N, 256), jnp.float32),
        ],
    )(onum, oden, fea, glob)


def _cry_a_body(fea_ref, rxn_ref, Ws_ref, b1_ref, W2_ref, b2_ref,
                gates_o, gmax_o, gmax_acc):
    i = pl.program_id(0)

    @pl.when(i == 0)
    def _():
        gmax_acc[...] = jnp.full((HEADS, C), NEG, jnp.float32)

    h = _leaky(jnp.dot(fea_ref[...], Ws_ref[...], preferred_element_type=jnp.float32)
               + b1_ref[...])
    g = jnp.dot(h, W2_ref[...], preferred_element_type=jnp.float32) + b2_ref[...]
    gates_o[...] = g
    rxn = rxn_ref[0, 0, :]
    z = rxn[:, None] == jax.lax.broadcasted_iota(jnp.int32, (BN, C), 1)
    for hh in range(HEADS):
        masked = jnp.where(z, g[:, hh:hh + 1], NEG)
        gmax_acc[hh, :] = jnp.maximum(gmax_acc[hh, :], jnp.max(masked, axis=0))

    @pl.when(i == pl.num_programs(0) - 1)
    def _():
        gmax_o[...] = gmax_acc[...]


def _cry_b_body(fea_ref, pw_ref, rxn_ref, gates_ref, gmax_ref,
                Ws_ref, b1_ref, W2_ref, b2_ref, x_o, acc):
    i = pl.program_id(0)

    @pl.when(i == 0)
    def _():
        acc[...] = jnp.zeros((HEADS, C, FEA + 16), jnp.float32)

    h = _leaky(jnp.dot(fea_ref[...], Ws_ref[...], preferred_element_type=jnp.float32)
               + b1_ref[...])
    rxn = rxn_ref[0, 0, :]
    z = rxn[:, None] == jax.lax.broadcasted_iota(jnp.int32, (BN, C), 1)
    zf = z.astype(jnp.float32)
    pw = pw_ref[...]
    for hh in range(HEADS):
        gm = jnp.max(jnp.where(z, gmax_ref[hh, :][None, :], NEG), axis=1)
        e = pw[:, 0] * jnp.exp(gates_ref[:, hh] - gm)
        m = jnp.dot(h[:, hh * HID:(hh + 1) * HID], W2_ref[hh],
                    preferred_element_type=jnp.float32) + b2_ref[hh]
        val = jnp.concatenate(
            [m * e[:, None], e[:, None], jnp.zeros((BN, 15), jnp.float32)], axis=1)
        acc[hh] += jax.lax.dot_general(zf, val, (((0,), (0,)), ((), ())),
                                       preferred_element_type=jnp.float32)

    @pl.when(i == pl.num_programs(0) - 1)
    def _():
        x = jnp.zeros((C, FEA), jnp.float32)
        for hh in range(HEADS):
            a = acc[hh]
            x += a[:, :FEA] / (a[:, FEA:FEA + 1] + 1e-13)
        x_o[...] = x * (1.0 / HEADS)


def _cry(fea, pw, rxn, cry_params):
    Ws = jnp.concatenate([cry_params[h]["gate"]["layers"][0][0] for h in range(HEADS)], axis=1)
    b1 = jnp.concatenate([cry_params[h]["gate"]["layers"][0][1] for h in range(HEADS)])
    W2 = jnp.zeros((HEADS * HID, HEADS), jnp.float32)
    for h in range(HEADS):
        W2 = W2.at[h * HID:(h + 1) * HID, h].set(cry_params[h]["gate"]["out"][0][:, 0])
    b2 = jnp.stack([cry_params[h]["gate"]["out"][1][0] for h in range(HEADS)])
    Wsm = jnp.concatenate([cry_params[h]["msg"]["layers"][0][0] for h in range(HEADS)], axis=1)
    b1m = jnp.concatenate([cry_params[h]["msg"]["layers"][0][1] for h in range(HEADS)])
    W2m = jnp.stack([cry_params[h]["msg"]["out"][0] for h in range(HEADS)])
    b2m = jnp.stack([cry_params[h]["msg"]["out"][1].reshape(1, FEA) for h in range(HEADS)])

    rxn3 = rxn.reshape(N // BN, 1, BN)
    grid = (N // BN,)
    gates, gmax = pl.pallas_call(
        _cry_a_body,
        grid=grid,
        in_specs=[
            pl.BlockSpec((BN, FEA), lambda i: (i, 0)),
            pl.BlockSpec((1, 1, BN), lambda i: (i, 0, 0)),
            pl.BlockSpec((FEA, HEADS * HID), lambda i: (0, 0)),
            pl.BlockSpec((1, HEADS * HID), lambda i: (0, 0)),
            pl.BlockSpec((HEADS * HID, HEADS), lambda i: (0, 0)),
            pl.BlockSpec((1, HEADS), lambda i: (0, 0)),
        ],
        out_specs=[
            pl.BlockSpec((BN, HEADS), lambda i: (i, 0)),
            pl.BlockSpec((HEADS, C), lambda i: (0, 0)),
        ],
        out_shape=[
            jax.ShapeDtypeStruct((N, HEADS), jnp.float32),
            jax.ShapeDtypeStruct((HEADS, C), jnp.float32),
        ],
        scratch_shapes=[pltpu.VMEM((HEADS, C), jnp.float32)],
    )(fea, rxn3, Ws, b1.reshape(1, -1), W2, b2.reshape(1, -1))

    x = pl.pallas_call(
        _cry_b_body,
        grid=grid,
        in_specs=[
            pl.BlockSpec((BN, FEA), lambda i: (i, 0)),
            pl.BlockSpec((BN, 1), lambda i: (i, 0)),
            pl.BlockSpec((1, 1, BN), lambda i: (i, 0, 0)),
            pl.BlockSpec((BN, HEADS), lambda i: (i, 0)),
            pl.BlockSpec((HEADS, C), lambda i: (0, 0)),
            pl.BlockSpec((FEA, HEADS * HID), lambda i: (0, 0)),
            pl.BlockSpec((1, HEADS * HID), lambda i: (0, 0)),
            pl.BlockSpec((HEADS, HID, FEA), lambda i: (0, 0, 0)),
            pl.BlockSpec((HEADS, 1, FEA), lambda i: (0, 0, 0)),
        ],
        out_specs=pl.BlockSpec((C, FEA), lambda i: (0, 0)),
        out_shape=jax.ShapeDtypeStruct((C, FEA), jnp.float32),
        scratch_shapes=[pltpu.VMEM((HEADS, C, FEA + 16), jnp.float32)],
    )(fea, pw, rxn3, gates, gmax, Wsm, b1m.reshape(1, -1), W2m, b2m)
    return x


def _res_mlp_kernel(x_ref, *refs):
    out_ref = refs[-1]
    ws = refs[:-1]
    x = x_ref[...]
    i = 0
    for li, has_res in enumerate((True, True, False, True)):
        W = ws[i][...]
        b = ws[i + 1][...]
        i += 2
        if has_res:
            Wres = ws[i][...]
            i += 1
            res = jnp.dot(x, Wres, preferred_element_type=jnp.float32)
        else:
            res = x
        x = jax.nn.relu(jnp.dot(x, W, preferred_element_type=jnp.float32) + b) + res
    Wo = ws[i][...]
    bo = ws[i + 1][...]
    out_ref[...] = jnp.dot(x, Wo, preferred_element_type=jnp.float32) + bo


def _res_mlp(x, res_layers, out_wb):
    CP = 2048
    xp = jnp.zeros((CP, x.shape[1]), jnp.float32).at[: x.shape[0]].set(x)
    args = []
    for (W, b, Wres) in res_layers:
        args.append(W)
        args.append(b.reshape(1, -1))
        if Wres is not None:
            args.append(Wres)
    Wo, bo = out_wb
    args.append(Wo)
    args.append(bo.reshape(1, -1))
    out = pl.pallas_call(
        _res_mlp_kernel,
        out_shape=jax.ShapeDtypeStruct((CP, TGT), jnp.float32),
    )(xp, *args)
    return out[: x.shape[0]]


_SC_MESH = plsc.VectorSubcoreMesh(core_axis_name="c", subcore_axis_name="s")
_SC_CP = pltpu.CompilerParams()
if "needs_layout_passes" in pltpu.CompilerParams.__dataclass_fields__:
    _SC_CP = dataclasses.replace(_SC_CP, needs_layout_passes=False)
NWORK = 32
GCH = 256
NCHG = M // GCH
CH2 = 640
NCH2 = M // CH2
ACH = 128
NCH3 = M // 2 // ACH


def _sc_gather_body(tself_hbm, fea_hbm, sidx_hbm, nidx_hbm, es_hbm, en_hbm,
                    ibs, ibn, esb, enb, sems):
    wid = jax.lax.axis_index("s") * 2 + jax.lax.axis_index("c")

    @pl.loop(wid, NCHG, step=NWORK)
    def _(k):
        base = k * GCH
        d1 = pltpu.async_copy(sidx_hbm.at[pl.ds(base, GCH)], ibs, sems.at[0])
        d2 = pltpu.async_copy(nidx_hbm.at[pl.ds(base, GCH)], ibn, sems.at[1])
        d1.wait()
        d2.wait()
        dd = []
        for j in range(2):
            dd.append(pltpu.async_copy(tself_hbm.at[ibs.at[pl.ds(j * 128, 128)]],
                                       esb.at[pl.ds(j * 128, 128)], sems.at[2]))
            dd.append(pltpu.async_copy(fea_hbm.at[ibn.at[pl.ds(j * 128, 128)]],
                                       enb.at[pl.ds(j * 128, 128)], sems.at[3]))
        for d in dd:
            d.wait()
        d5 = pltpu.async_copy(esb, es_hbm.at[pl.ds(k * GCH, GCH)], sems.at[4])
        d6 = pltpu.async_copy(enb, en_hbm.at[pl.ds(k * GCH, GCH)], sems.at[5])
        d5.wait()
        d6.wait()


def _sc_gather(tself, fea, sidx2, nidx2):
    f = pl.kernel(
        _sc_gather_body,
        out_type=(jax.ShapeDtypeStruct((M, 256), jnp.float32),
                  jax.ShapeDtypeStruct((M, FEA), jnp.float32)),
        mesh=_SC_MESH,
        scratch_types=[
            pltpu.VMEM((GCH,), jnp.int32),
            pltpu.VMEM((GCH,), jnp.int32),
            pltpu.VMEM((GCH, 256), jnp.float32),
            pltpu.VMEM((GCH, FEA), jnp.float32),
            pltpu.SemaphoreType.DMA((6,)),
        ],
    )
    return f(tself, fea, sidx2, nidx2)


def _sc_segmax_body(gates_hbm, sidx_hbm, out_hbm, acc, ib, gb, sems):
    wid = jax.lax.axis_index("s") * 2 + jax.lax.axis_index("c")

    @pl.loop(0, HEADS * N, step=16)
    def _(i):
        acc[pl.ds(i, 16)] = jnp.full((16,), NEG, jnp.float32)

    @pl.loop(wid, NCH2, step=NWORK)
    def _(k):
        base = k * CH2
        d1 = pltpu.async_copy(sidx_hbm.at[pl.ds(base, CH2)], ib, sems.at[0])
        d2 = pltpu.async_copy(gates_hbm.at[:, pl.ds(base, CH2)], gb, sems.at[1])
        d1.wait()
        d2.wait()

        @pl.loop(0, CH2 // 16)
        def _(v):
            idx = ib[pl.ds(v * 16, 16)]
            for h in range(HEADS):
                g = gb[h, pl.ds(v * 16, 16)]
                idx3 = idx + h * N

                @pl.loop(0, 16)
                def _(r):
                    cur = plsc.load_gather(acc, [idx3])
                    plsc.store_scatter(acc, [idx3], g, mask=g > cur)

    pltpu.sync_copy(acc, out_hbm.at[wid, 0])


def _sc_segmax(gates, sidx1):
    f = pl.kernel(
        _sc_segmax_body,
        out_type=jax.ShapeDtypeStruct((NWORK, 1, HEADS * N), jnp.float32),
        mesh=_SC_MESH,
        scratch_types=[
            pltpu.VMEM((HEADS * N,), jnp.float32),
            pltpu.VMEM((CH2,), jnp.int32),
            pltpu.VMEM((HEADS, CH2), jnp.float32),
            pltpu.SemaphoreType.DMA((2,)),
        ],
        compiler_params=_SC_CP,
    )
    return f(gates, sidx1)


def _maxmerge_body(p_ref, o_ref):
    o_ref[...] = jnp.max(p_ref[:, 0, :], axis=0, keepdims=True)


def _maxmerge(partials):
    out = pl.pallas_call(
        _maxmerge_body,
        out_shape=jax.ShapeDtypeStruct((1, HEADS * N), jnp.float32),
    )(partials)
    return out.reshape(HEADS * N)


def _sc_weights_body(gates_hbm, gmax_hbm, pw_hbm, sidx_hbm, nidx_hbm, w_hbm,
                     gmb, pwb, ibs, ibn, gb, wb, sems):
    wid = jax.lax.axis_index("s") * 2 + jax.lax.axis_index("c")
    pltpu.sync_copy(gmax_hbm, gmb)
    pltpu.sync_copy(pw_hbm, pwb)

    @pl.loop(wid, NCH2, step=NWORK)
    def _(k):
        base = k * CH2
        d1 = pltpu.async_copy(sidx_hbm.at[pl.ds(base, CH2)], ibs, sems.at[0])
        d2 = pltpu.async_copy(nidx_hbm.at[pl.ds(base, CH2)], ibn, sems.at[1])
        d3 = pltpu.async_copy(gates_hbm.at[:, pl.ds(base, CH2)], gb, sems.at[2])
        d1.wait()
        d2.wait()
        d3.wait()

        @pl.loop(0, CH2 // 16)
        def _(v):
            s16 = ibs[pl.ds(v * 16, 16)]
            n16 = ibn[pl.ds(v * 16, 16)]
            pwv = plsc.load_gather(pwb, [n16])
            for h in range(HEADS):
                g = gb[h, pl.ds(v * 16, 16)]
                m = plsc.load_gather(gmb, [s16 + h * N])
                wb[h, pl.ds(v * 16, 16)] = pwv * jnp.exp(g - m)

        d4 = pltpu.async_copy(wb, w_hbm.at[:, pl.ds(base, CH2)], sems.at[3])
        d4.wait()


def _sc_weights(gates, gmax1, pw1, sidx1, nidx1):
    f = pl.kernel(
        _sc_weights_body,
        out_type=jax.ShapeDtypeStruct((HEADS, M), jnp.float32),
        mesh=_SC_MESH,
        scratch_types=[
            pltpu.VMEM((HEADS * N,), jnp.float32),
            pltpu.VMEM((N,), jnp.float32),
            pltpu.VMEM((CH2,), jnp.int32),
            pltpu.VMEM((CH2,), jnp.int32),
            pltpu.VMEM((HEADS, CH2), jnp.float32),
            pltpu.VMEM((HEADS, CH2), jnp.float32),
            pltpu.SemaphoreType.DMA((4,)),
        ],
        compiler_params=_SC_CP,
    )
    return f(gates, gmax1, pw1, sidx1, nidx1)


def _sc_acc_body(wmsg_hbm, wden_hbm, sidx_hbm, zeros_hbm, onum_hbm, oden_hbm,
                 accs, ib, rb, sems):
    c = jax.lax.axis_index("c")
    s = jax.lax.axis_index("s")
    for h in range(HEADS + 1):
        @pl.when(s == 0)
        def _():
            pltpu.sync_copy(zeros_hbm, accs)

        plsc.subcore_barrier()

        @pl.loop(s, NCH3, step=16)
        def _(k):
            row = c * NCH3 + k
            d1 = pltpu.async_copy(sidx_hbm.at[pl.ds(row * ACH, ACH)], ib, sems.at[0])
            if h < HEADS:
                src = wmsg_hbm.at[h, pl.ds(row * ACH, ACH)]
            else:
                src = wden_hbm.at[pl.ds(row * ACH, ACH)]
            d2 = pltpu.async_copy(src, rb, sems.at[1])
            d1.wait()
            d2.wait()
            pltpu.sync_copy(rb, accs.at[ib], add=True)

        plsc.subcore_barrier()

        @pl.when(s == 0)
        def _():
            if h < HEADS:
                pltpu.sync_copy(accs, onum_hbm.at[c, h])
            else:
                pltpu.sync_copy(accs, oden_hbm.at[c])

        plsc.subcore_barrier()


def _sc_acc(wmsg, wden, sidx1, zeros128):
    f = pl.kernel(
        _sc_acc_body,
        out_type=(jax.ShapeDtypeStruct((2, HEADS, N, FEA), jnp.float32),
                  jax.ShapeDtypeStruct((2, N, FEA), jnp.float32)),
        mesh=_SC_MESH,
        scratch_types=[
            pltpu.VMEM_SHARED((N, FEA), jnp.float32),
            pltpu.VMEM((128,), jnp.int32),
            pltpu.VMEM((ACH, FEA), jnp.float32),
            pltpu.SemaphoreType.DMA((2,)),
        ],
    )
    return f(wmsg, wden, sidx1, zeros128)


def _pack_layer(heads):
    def cat(key, part):
        return jnp.concatenate([heads[h][key]["layers"][0][0][part] for h in range(HEADS)], axis=1)
    sl_self = slice(0, FEA)
    sl_nbr = slice(FEA, 2 * FEA)
    sl_glob = slice(2 * FEA, 2 * FEA + LAT)
    Wg_s = jnp.concatenate([cat("gate", sl_self), cat("gate", sl_glob)], axis=0)
    Wg_n = cat("gate", sl_nbr)
    b1g = jnp.concatenate([heads[h]["gate"]["layers"][0][1] for h in range(HEADS)])
    W2g = jnp.zeros((HEADS * HID, HEADS), jnp.float32)
    for h in range(HEADS):
        W2g = W2g.at[h * HID:(h + 1) * HID, h].set(heads[h]["gate"]["out"][0][:, 0])
    b2g = jnp.stack([heads[h]["gate"]["out"][1][0] for h in range(HEADS)])
    Wm_s = jnp.concatenate([cat("msg", sl_self), cat("msg", sl_glob)], axis=0)
    Wm_n = cat("msg", sl_nbr)
    b1m = jnp.concatenate([heads[h]["msg"]["layers"][0][1] for h in range(HEADS)])
    W2m = jnp.stack([heads[h]["msg"]["out"][0] for h in range(HEADS)])
    b2m = jnp.stack([heads[h]["msg"]["out"][1].reshape(1, FEA) for h in range(HEADS)])
    return (Wg_s, Wg_n, b1g.reshape(1, -1), W2g, b2g.reshape(1, -1),
            Wm_s, Wm_n, b1m.reshape(1, -1), W2m, b2m)


def kernel(prec_weights, orig_prec_fea, self_fea_idx, nbr_fea_idx, reaction_prec_idx, actions, params):
    pw = prec_weights
    rxn = reaction_prec_idx
    fea, glob, tself = _embed(orig_prec_fea, pw, rxn, params["embed"], actions)

    sidx1 = self_fea_idx.astype(jnp.int32)
    nidx1 = nbr_fea_idx.astype(jnp.int32)
    pw1 = pw.reshape(N)
    zeros128 = jnp.zeros((N, FEA), jnp.float32)
    for heads in params["graphs"]:
        (Wg_s, Wg_n, b1g, W2g, b2g,
         Wm_s, Wm_n, b1m, W2m, b2m) = _pack_layer(heads)
        es, en = _sc_gather(tself, fea, sidx1, nidx1)
        gates = _gates(es, en, Wg_s, Wg_n, b1g, W2g, b2g)
        partials = _sc_segmax(gates, sidx1)
        gmax1 = _maxmerge(partials)
        w = _sc_weights(gates, gmax1, pw1, sidx1, nidx1)
        wmsg, wden = _msgs(es, en, w, Wm_s, Wm_n, b1m, W2m, b2m)
        onum, oden = _sc_acc(wmsg, wden, sidx1, zeros128)
        fea, tself = _norm(onum, oden, fea, glob)

    x = _cry(fea, pw, rxn, params["cry"])
    return _res_mlp(x, params["res"], params["out"])


# speedup vs baseline: 7.0057x; 1.0924x over previous
import dataclasses
import functools

import jax
import jax.numpy as jnp
from jax.experimental import pallas as pl
from jax.experimental.pallas import tpu as pltpu
from jax.experimental.pallas import tpu_sc as plsc

N = 10000
M = 160000
C = 2000
ORIG = 200
FEA = 128
LAT = 32
HEADS = 3
TGT = 64
HID = 256

BM = 640
BN = 400
NEG = -3.0e38


def _leaky(x):
    return jnp.where(x > 0, x, 0.01 * x)


def _embed_body(orig_ref, pw_ref, rxn_ref, Wemb_ref, act_ref, fea_ref, glob_ref, tself_ref):
    orig = orig_ref[...]
    emb = jnp.dot(orig, Wemb_ref[...], preferred_element_type=jnp.float32)
    col = jax.lax.broadcasted_iota(jnp.int32, (BN, FEA), 1)
    fea = jnp.where(col == FEA - 1, pw_ref[...], emb)
    fea_ref[...] = fea
    rxn = rxn_ref[0, 0, :]
    z = (rxn[:, None] == jax.lax.broadcasted_iota(jnp.int32, (BN, C), 1)).astype(jnp.float32)
    glob = jax.lax.dot_general(z, act_ref[...], (((1,), (0,)), ((), ())),
                               preferred_element_type=jnp.float32)
    glob_ref[...] = glob
    tself_ref[...] = jnp.concatenate(
        [fea, glob, jnp.zeros((BN, 96), jnp.float32)], axis=1)


def _embed(orig, pw, rxn, Wemb, actions):
    Wpad = jnp.concatenate([Wemb, jnp.zeros((ORIG, 1), jnp.float32)], axis=1)
    rxn3 = rxn.reshape(N // BN, 1, BN)
    grid = (N // BN,)
    return pl.pallas_call(
        _embed_body,
        grid=grid,
        in_specs=[
            pl.BlockSpec((BN, ORIG), lambda i: (i, 0)),
            pl.BlockSpec((BN, 1), lambda i: (i, 0)),
            pl.BlockSpec((1, 1, BN), lambda i: (i, 0, 0)),
            pl.BlockSpec((ORIG, FEA), lambda i: (0, 0)),
            pl.BlockSpec((C, LAT), lambda i: (0, 0)),
        ],
        out_specs=[
            pl.BlockSpec((BN, FEA), lambda i: (i, 0)),
            pl.BlockSpec((BN, LAT), lambda i: (i, 0)),
            pl.BlockSpec((BN, 256), lambda i: (i, 0)),
        ],
        out_shape=[
            jax.ShapeDtypeStruct((N, FEA), jnp.float32),
            jax.ShapeDtypeStruct((N, LAT), jnp.float32),
            jax.ShapeDtypeStruct((N, 256), jnp.float32),
        ],
    )(orig, pw, rxn3, Wpad, actions)


def _gates_body(es_ref, en_ref, Ws_ref, Wn_ref, b1_ref, W2_ref, b2_ref, out_ref):
    h = jnp.dot(es_ref[:, :160], Ws_ref[...], preferred_element_type=jnp.float32)
    h += jnp.dot(en_ref[...], Wn_ref[...], preferred_element_type=jnp.float32)
    h = _leaky(h + b1_ref[...])
    g = jnp.dot(h, W2_ref[...], preferred_element_type=jnp.float32) + b2_ref[...]
    out_ref[...] = g.T


def _gates(es, en, Ws, Wn, b1, W2, b2):
    grid = (M // BM,)
    return pl.pallas_call(
        _gates_body,
        grid=grid,
        in_specs=[
            pl.BlockSpec((BM, 256), lambda i: (i, 0)),
            pl.BlockSpec((BM, FEA), lambda i: (i, 0)),
            pl.BlockSpec((160, HEADS * HID), lambda i: (0, 0)),
            pl.BlockSpec((FEA, HEADS * HID), lambda i: (0, 0)),
            pl.BlockSpec((1, HEADS * HID), lambda i: (0, 0)),
            pl.BlockSpec((HEADS * HID, HEADS), lambda i: (0, 0)),
            pl.BlockSpec((1, HEADS), lambda i: (0, 0)),
        ],
        out_specs=pl.BlockSpec((HEADS, BM), lambda i: (0, i)),
        out_shape=jax.ShapeDtypeStruct((HEADS, M), jnp.float32),
    )(es, en, Ws, Wn, b1, W2, b2)


def _msgs_body(es_ref, en_ref, w_ref, Ws_ref, Wn_ref, b1_ref, W2_ref, b2_ref, out_ref, den_ref):
    h = jnp.dot(es_ref[:, :160], Ws_ref[...], preferred_element_type=jnp.float32)
    h += jnp.dot(en_ref[...], Wn_ref[...], preferred_element_type=jnp.float32)
    h = _leaky(h + b1_ref[...])
    dens = []
    for hh in range(HEADS):
        m = jnp.dot(h[:, hh * HID:(hh + 1) * HID], W2_ref[hh],
                    preferred_element_type=jnp.float32) + b2_ref[hh]
        wv = w_ref[hh][:, None]
        out_ref[hh] = m * wv
        dens.append(jnp.broadcast_to(wv, (BM, 16)))
    den_ref[...] = jnp.concatenate(
        dens + [jnp.zeros((BM, 128 - 16 * HEADS), jnp.float32)], axis=1)


def _msgs(es, en, w, Ws, Wn, b1, W2, b2):
    grid = (M // BM,)
    return pl.pallas_call(
        _msgs_body,
        grid=grid,
        in_specs=[
            pl.BlockSpec((BM, 256), lambda i: (i, 0)),
            pl.BlockSpec((BM, FEA), lambda i: (i, 0)),
            pl.BlockSpec((HEADS, BM), lambda i: (0, i)),
            pl.BlockSpec((160, HEADS * HID), lambda i: (0, 0)),
            pl.BlockSpec((FEA, HEADS * HID), lambda i: (0, 0)),
            pl.BlockSpec((1, HEADS * HID), lambda i: (0, 0)),
            pl.BlockSpec((HEADS, HID, FEA), lambda i: (0, 0, 0)),
            pl.BlockSpec((HEADS, 1, FEA), lambda i: (0, 0, 0)),
        ],
        out_specs=[
            pl.BlockSpec((HEADS, BM, FEA), lambda i: (0, i, 0)),
            pl.BlockSpec((BM, FEA), lambda i: (i, 0)),
        ],
        out_shape=[
            jax.ShapeDtypeStruct((HEADS, M, FEA), jnp.float32),
            jax.ShapeDtypeStruct((M, FEA), jnp.float32),
        ],
    )(es, en, w, Ws, Wn, b1, W2, b2)


def _norm_body(num_ref, den_ref, fea_ref, glob_ref, fea_o, tself_o):
    av = num_ref[0] + num_ref[1]
    denp = den_ref[0] + den_ref[1]
    acc = jnp.zeros((BN, FEA), jnp.float32)
    for hh in range(HEADS):
        acc += av[hh] / (denp[:, 16 * hh:16 * hh + 1] + 1e-13)
    fea = acc * (1.0 / HEADS) + fea_ref[...]
    fea_o[...] = fea
    tself_o[...] = jnp.concatenate(
        [fea, glob_ref[...], jnp.zeros((BN, 96), jnp.float32)], axis=1)


def _norm(onum, oden, fea, glob):
    grid = (N // BN,)
    return pl.pallas_call(
        _norm_body,
        grid=grid,
        in_specs=[
            pl.BlockSpec((2, HEADS, BN, FEA), lambda i: (0, 0, i, 0)),
            pl.BlockSpec((2, BN, FEA), lambda i: (0, i, 0)),
            pl.BlockSpec((BN, FEA), lambda i: (i, 0)),
            pl.BlockSpec((BN, LAT), lambda i: (i, 0)),
        ],
        out_specs=[
            pl.BlockSpec((BN, FEA), lambda i: (i, 0)),
            pl.BlockSpec((BN, 256), lambda i: (i, 0)),
        ],
        out_shape=[
            jax.ShapeDtypeStruct((N, FEA), jnp.float32),
            jax.ShapeDtypeStruct((N, 256), jnp.float32),
        ],
    )(onum, oden, fea, glob)


def _cry_a_body(fea_ref, rxn_ref, Ws_ref, b1_ref, W2_ref, b2_ref,
                gates_o, gmax_o, gmax_acc):
    i = pl.program_id(0)

    @pl.when(i == 0)
    def _():
        gmax_acc[...] = jnp.full((HEADS, C), NEG, jnp.float32)

    h = _leaky(jnp.dot(fea_ref[...], Ws_ref[...], preferred_element_type=jnp.float32)
               + b1_ref[...])
    g = jnp.dot(h, W2_ref[...], preferred_element_type=jnp.float32) + b2_ref[...]
    gates_o[...] = g
    rxn = rxn_ref[0, 0, :]
    z = rxn[:, None] == jax.lax.broadcasted_iota(jnp.int32, (BN, C), 1)
    for hh in range(HEADS):
        masked = jnp.where(z, g[:, hh:hh + 1], NEG)
        gmax_acc[hh, :] = jnp.maximum(gmax_acc[hh, :], jnp.max(masked, axis=0))

    @pl.when(i == pl.num_programs(0) - 1)
    def _():
        gmax_o[...] = gmax_acc[...]


def _cry_b_body(fea_ref, pw_ref, rxn_ref, gates_ref, gmax_ref,
                Ws_ref, b1_ref, W2_ref, b2_ref, x_o, acc):
    i = pl.program_id(0)

    @pl.when(i == 0)
    def _():
        acc[...] = jnp.zeros((HEADS, C, FEA + 16), jnp.float32)

    h = _leaky(jnp.dot(fea_ref[...], Ws_ref[...], preferred_element_type=jnp.float32)
               + b1_ref[...])
    rxn = rxn_ref[0, 0, :]
    z = rxn[:, None] == jax.lax.broadcasted_iota(jnp.int32, (BN, C), 1)
    zf = z.astype(jnp.float32)
    pw = pw_ref[...]
    for hh in range(HEADS):
        gm = jnp.max(jnp.where(z, gmax_ref[hh, :][None, :], NEG), axis=1)
        e = pw[:, 0] * jnp.exp(gates_ref[:, hh] - gm)
        m = jnp.dot(h[:, hh * HID:(hh + 1) * HID], W2_ref[hh],
                    preferred_element_type=jnp.float32) + b2_ref[hh]
        val = jnp.concatenate(
            [m * e[:, None], e[:, None], jnp.zeros((BN, 15), jnp.float32)], axis=1)
        acc[hh] += jax.lax.dot_general(zf, val, (((0,), (0,)), ((), ())),
                                       preferred_element_type=jnp.float32)

    @pl.when(i == pl.num_programs(0) - 1)
    def _():
        x = jnp.zeros((C, FEA), jnp.float32)
        for hh in range(HEADS):
            a = acc[hh]
            x += a[:, :FEA] / (a[:, FEA:FEA + 1] + 1e-13)
        x_o[...] = x * (1.0 / HEADS)


def _cry(fea, pw, rxn, cry_params):
    Ws = jnp.concatenate([cry_params[h]["gate"]["layers"][0][0] for h in range(HEADS)], axis=1)
    b1 = jnp.concatenate([cry_params[h]["gate"]["layers"][0][1] for h in range(HEADS)])
    W2 = jnp.zeros((HEADS * HID, HEADS), jnp.float32)
    for h in range(HEADS):
        W2 = W2.at[h * HID:(h + 1) * HID, h].set(cry_params[h]["gate"]["out"][0][:, 0])
    b2 = jnp.stack([cry_params[h]["gate"]["out"][1][0] for h in range(HEADS)])
    Wsm = jnp.concatenate([cry_params[h]["msg"]["layers"][0][0] for h in range(HEADS)], axis=1)
    b1m = jnp.concatenate([cry_params[h]["msg"]["layers"][0][1] for h in range(HEADS)])
    W2m = jnp.stack([cry_params[h]["msg"]["out"][0] for h in range(HEADS)])
    b2m = jnp.stack([cry_params[h]["msg"]["out"][1].reshape(1, FEA) for h in range(HEADS)])

    rxn3 = rxn.reshape(N // BN, 1, BN)
    grid = (N // BN,)
    gates, gmax = pl.pallas_call(
        _cry_a_body,
        grid=grid,
        in_specs=[
            pl.BlockSpec((BN, FEA), lambda i: (i, 0)),
            pl.BlockSpec((1, 1, BN), lambda i: (i, 0, 0)),
            pl.BlockSpec((FEA, HEADS * HID), lambda i: (0, 0)),
            pl.BlockSpec((1, HEADS * HID), lambda i: (0, 0)),
            pl.BlockSpec((HEADS * HID, HEADS), lambda i: (0, 0)),
            pl.BlockSpec((1, HEADS), lambda i: (0, 0)),
        ],
        out_specs=[
            pl.BlockSpec((BN, HEADS), lambda i: (i, 0)),
            pl.BlockSpec((HEADS, C), lambda i: (0, 0)),
        ],
        out_shape=[
            jax.ShapeDtypeStruct((N, HEADS), jnp.float32),
            jax.ShapeDtypeStruct((HEADS, C), jnp.float32),
        ],
        scratch_shapes=[pltpu.VMEM((HEADS, C), jnp.float32)],
    )(fea, rxn3, Ws, b1.reshape(1, -1), W2, b2.reshape(1, -1))

    x = pl.pallas_call(
        _cry_b_body,
        grid=grid,
        in_specs=[
            pl.BlockSpec((BN, FEA), lambda i: (i, 0)),
            pl.BlockSpec((BN, 1), lambda i: (i, 0)),
            pl.BlockSpec((1, 1, BN), lambda i: (i, 0, 0)),
            pl.BlockSpec((BN, HEADS), lambda i: (i, 0)),
            pl.BlockSpec((HEADS, C), lambda i: (0, 0)),
            pl.BlockSpec((FEA, HEADS * HID), lambda i: (0, 0)),
            pl.BlockSpec((1, HEADS * HID), lambda i: (0, 0)),
            pl.BlockSpec((HEADS, HID, FEA), lambda i: (0, 0, 0)),
            pl.BlockSpec((HEADS, 1, FEA), lambda i: (0, 0, 0)),
        ],
        out_specs=pl.BlockSpec((C, FEA), lambda i: (0, 0)),
        out_shape=jax.ShapeDtypeStruct((C, FEA), jnp.float32),
        scratch_shapes=[pltpu.VMEM((HEADS, C, FEA + 16), jnp.float32)],
    )(fea, pw, rxn3, gates, gmax, Wsm, b1m.reshape(1, -1), W2m, b2m)
    return x


def _res_mlp_kernel(x_ref, *refs):
    out_ref = refs[-1]
    ws = refs[:-1]
    x = x_ref[...]
    i = 0
    for li, has_res in enumerate((True, True, False, True)):
        W = ws[i][...]
        b = ws[i + 1][...]
        i += 2
        if has_res:
            Wres = ws[i][...]
            i += 1
            res = jnp.dot(x, Wres, preferred_element_type=jnp.float32)
        else:
            res = x
        x = jax.nn.relu(jnp.dot(x, W, preferred_element_type=jnp.float32) + b) + res
    Wo = ws[i][...]
    bo = ws[i + 1][...]
    out_ref[...] = jnp.dot(x, Wo, preferred_element_type=jnp.float32) + bo


def _res_mlp(x, res_layers, out_wb):
    CP = 2048
    xp = jnp.zeros((CP, x.shape[1]), jnp.float32).at[: x.shape[0]].set(x)
    args = []
    for (W, b, Wres) in res_layers:
        args.append(W)
        args.append(b.reshape(1, -1))
        if Wres is not None:
            args.append(Wres)
    Wo, bo = out_wb
    args.append(Wo)
    args.append(bo.reshape(1, -1))
    out = pl.pallas_call(
        _res_mlp_kernel,
        out_shape=jax.ShapeDtypeStruct((CP, TGT), jnp.float32),
    )(xp, *args)
    return out[: x.shape[0]]


_SC_MESH = plsc.VectorSubcoreMesh(core_axis_name="c", subcore_axis_name="s")
_SC_CP = pltpu.CompilerParams()
if "needs_layout_passes" in pltpu.CompilerParams.__dataclass_fields__:
    _SC_CP = dataclasses.replace(_SC_CP, needs_layout_passes=False)
NWORK = 32
GCH = 256
NCHG = M // GCH
CH2 = 640
NCH2 = M // CH2
ACH = 128
NCH3 = M // 2 // ACH


def _sc_gather_body(tself_hbm, fea_hbm, sidx_hbm, nidx_hbm, es_hbm, en_hbm,
                    ibs, ibn, esb, enb, sems):
    wid = jax.lax.axis_index("s") * 2 + jax.lax.axis_index("c")

    @pl.loop(wid, NCHG, step=NWORK)
    def _(k):
        base = k * GCH
        d1 = pltpu.async_copy(sidx_hbm.at[pl.ds(base, GCH)], ibs, sems.at[0])
        d2 = pltpu.async_copy(nidx_hbm.at[pl.ds(base, GCH)], ibn, sems.at[1])
        d1.wait()
        d2.wait()
        dd = []
        for j in range(2):
            dd.append(pltpu.async_copy(tself_hbm.at[ibs.at[pl.ds(j * 128, 128)]],
                                       esb.at[pl.ds(j * 128, 128)], sems.at[2]))
            dd.append(pltpu.async_copy(fea_hbm.at[ibn.at[pl.ds(j * 128, 128)]],
                                       enb.at[pl.ds(j * 128, 128)], sems.at[3]))
        for d in dd:
            d.wait()
        d5 = pltpu.async_copy(esb, es_hbm.at[pl.ds(k * GCH, GCH)], sems.at[4])
        d6 = pltpu.async_copy(enb, en_hbm.at[pl.ds(k * GCH, GCH)], sems.at[5])
        d5.wait()
        d6.wait()


def _sc_gather(tself, fea, sidx2, nidx2):
    f = pl.kernel(
        _sc_gather_body,
        out_type=(jax.ShapeDtypeStruct((M, 256), jnp.float32),
                  jax.ShapeDtypeStruct((M, FEA), jnp.float32)),
        mesh=_SC_MESH,
        scratch_types=[
            pltpu.VMEM((GCH,), jnp.int32),
            pltpu.VMEM((GCH,), jnp.int32),
            pltpu.VMEM((GCH, 256), jnp.float32),
            pltpu.VMEM((GCH, FEA), jnp.float32),
            pltpu.SemaphoreType.DMA((6,)),
        ],
    )
    return f(tself, fea, sidx2, nidx2)


def _sc_segmax_body(gates_hbm, sidx_hbm, out_hbm, acc, ib, gb, sems):
    wid = jax.lax.axis_index("s") * 2 + jax.lax.axis_index("c")

    @pl.loop(0, HEADS * N, step=16)
    def _(i):
        acc[pl.ds(i, 16)] = jnp.full((16,), NEG, jnp.float32)

    @pl.loop(wid, NCH2, step=NWORK)
    def _(k):
        base = k * CH2
        d1 = pltpu.async_copy(sidx_hbm.at[pl.ds(base, CH2)], ib, sems.at[0])
        d2 = pltpu.async_copy(gates_hbm.at[:, pl.ds(base, CH2)], gb, sems.at[1])
        d1.wait()
        d2.wait()

        @pl.loop(0, CH2 // 16)
        def _(v):
            idx = ib[pl.ds(v * 16, 16)]
            for h in range(HEADS):
                g = gb[h, pl.ds(v * 16, 16)]
                idx3 = idx + h * N

                @pl.loop(0, 16)
                def _(r):
                    cur = plsc.load_gather(acc, [idx3])
                    plsc.store_scatter(acc, [idx3], g, mask=g > cur)

    pltpu.sync_copy(acc, out_hbm.at[wid, 0])


def _sc_segmax(gates, sidx1):
    f = pl.kernel(
        _sc_segmax_body,
        out_type=jax.ShapeDtypeStruct((NWORK, 1, HEADS * N), jnp.float32),
        mesh=_SC_MESH,
        scratch_types=[
            pltpu.VMEM((HEADS * N,), jnp.float32),
            pltpu.VMEM((CH2,), jnp.int32),
            pltpu.VMEM((HEADS, CH2), jnp.float32),
            pltpu.SemaphoreType.DMA((2,)),
        ],
        compiler_params=_SC_CP,
    )
    return f(gates, sidx1)


def _maxmerge_body(p_ref, o_ref):
    o_ref[...] = jnp.max(p_ref[:, 0, :], axis=0, keepdims=True)


def _maxmerge(partials):
    out = pl.pallas_call(
        _maxmerge_body,
        out_shape=jax.ShapeDtypeStruct((1, HEADS * N), jnp.float32),
    )(partials)
    return out.reshape(HEADS * N)


def _sc_weights_body(gates_hbm, gmax_hbm, pw_hbm, sidx_hbm, nidx_hbm, w_hbm,
                     gmb, pwb, ibs, ibn, gb, wb, sems):
    wid = jax.lax.axis_index("s") * 2 + jax.lax.axis_index("c")
    pltpu.sync_copy(gmax_hbm, gmb)
    pltpu.sync_copy(pw_hbm, pwb)

    @pl.loop(wid, NCH2, step=NWORK)
    def _(k):
        base = k * CH2
        d1 = pltpu.async_copy(sidx_hbm.at[pl.ds(base, CH2)], ibs, sems.at[0])
        d2 = pltpu.async_copy(nidx_hbm.at[pl.ds(base, CH2)], ibn, sems.at[1])
        d3 = pltpu.async_copy(gates_hbm.at[:, pl.ds(base, CH2)], gb, sems.at[2])
        d1.wait()
        d2.wait()
        d3.wait()

        @pl.loop(0, CH2 // 16)
        def _(v):
            s16 = ibs[pl.ds(v * 16, 16)]
            n16 = ibn[pl.ds(v * 16, 16)]
            pwv = plsc.load_gather(pwb, [n16])
            for h in range(HEADS):
                g = gb[h, pl.ds(v * 16, 16)]
                m = plsc.load_gather(gmb, [s16 + h * N])
                wb[h, pl.ds(v * 16, 16)] = pwv * jnp.exp(g - m)

        d4 = pltpu.async_copy(wb, w_hbm.at[:, pl.ds(base, CH2)], sems.at[3])
        d4.wait()


def _sc_weights(gates, gmax1, pw1, sidx1, nidx1):
    f = pl.kernel(
        _sc_weights_body,
        out_type=jax.ShapeDtypeStruct((HEADS, M), jnp.float32),
        mesh=_SC_MESH,
        scratch_types=[
            pltpu.VMEM((HEADS * N,), jnp.float32),
            pltpu.VMEM((N,), jnp.float32),
            pltpu.VMEM((CH2,), jnp.int32),
            pltpu.VMEM((CH2,), jnp.int32),
            pltpu.VMEM((HEADS, CH2), jnp.float32),
            pltpu.VMEM((HEADS, CH2), jnp.float32),
            pltpu.SemaphoreType.DMA((4,)),
        ],
        compiler_params=_SC_CP,
    )
    return f(gates, gmax1, pw1, sidx1, nidx1)


def _sc_acc_body(wmsg_hbm, wden_hbm, sidx_hbm, zeros_hbm, onum_hbm, oden_hbm,
                 accs, iba, ibb, rba, rbb, sems):
    c = jax.lax.axis_index("c")
    s = jax.lax.axis_index("s")
    zoff = s * 640

    n = (NCH3 - 1 - s) // 16 + 1
    nn = (n + 1) // 2

    for h in range(HEADS + 1):
        if h < HEADS:
            src_all = wmsg_hbm.at[h]
        else:
            src_all = wden_hbm

        @pl.when(s < 15)
        def _():
            pltpu.sync_copy(zeros_hbm.at[pl.ds(zoff, 640)], accs.at[pl.ds(zoff, 640)])

        @pl.when(s == 15)
        def _():
            pltpu.sync_copy(zeros_hbm.at[pl.ds(9600, 400)], accs.at[pl.ds(9600, 400)])

        plsc.subcore_barrier()

        def issue(j, ib, rb, sem):
            row = c * NCH3 + s + 16 * j
            pltpu.async_copy(sidx_hbm.at[pl.ds(row * ACH, ACH)], ib, sem)
            pltpu.async_copy(src_all.at[pl.ds(row * ACH, ACH)], rb, sem)

        def wait_and_scatter(ib, rb, sem):
            pltpu.make_async_copy(sidx_hbm.at[pl.ds(0, ACH)], ib, sem).wait()
            pltpu.make_async_copy(src_all.at[pl.ds(0, ACH)], rb, sem).wait()
            pltpu.sync_copy(rb, accs.at[ib], add=True)

        issue(0, iba, rba, sems.at[0])

        @pl.loop(0, nn)
        def _(jj):
            ja = 2 * jj
            jb = 2 * jj + 1

            @pl.when(jb < n)
            def _():
                issue(jb, ibb, rbb, sems.at[1])

            wait_and_scatter(iba, rba, sems.at[0])

            @pl.when(ja + 2 < n)
            def _():
                issue(ja + 2, iba, rba, sems.at[0])

            @pl.when(jb < n)
            def _():
                wait_and_scatter(ibb, rbb, sems.at[1])

        plsc.subcore_barrier()

        if h < HEADS:
            dst_all = onum_hbm.at[c, h]
        else:
            dst_all = oden_hbm.at[c]

        @pl.when(s < 15)
        def _():
            pltpu.sync_copy(accs.at[pl.ds(zoff, 640)], dst_all.at[pl.ds(zoff, 640)])

        @pl.when(s == 15)
        def _():
            pltpu.sync_copy(accs.at[pl.ds(9600, 400)], dst_all.at[pl.ds(9600, 400)])

        plsc.subcore_barrier()


def _sc_acc(wmsg, wden, sidx1, zeros128):
    f = pl.kernel(
        _sc_acc_body,
        out_type=(jax.ShapeDtypeStruct((2, HEADS, N, FEA), jnp.float32),
                  jax.ShapeDtypeStruct((2, N, FEA), jnp.float32)),
        mesh=_SC_MESH,
        scratch_types=[
            pltpu.VMEM_SHARED((N, FEA), jnp.float32),
            pltpu.VMEM((128,), jnp.int32),
            pltpu.VMEM((128,), jnp.int32),
            pltpu.VMEM((ACH, FEA), jnp.float32),
            pltpu.VMEM((ACH, FEA), jnp.float32),
            pltpu.SemaphoreType.DMA((2,)),
        ],
    )
    return f(wmsg, wden, sidx1, zeros128)


def _pack_layer(heads):
    def cat(key, part):
        return jnp.concatenate([heads[h][key]["layers"][0][0][part] for h in range(HEADS)], axis=1)
    sl_self = slice(0, FEA)
    sl_nbr = slice(FEA, 2 * FEA)
    sl_glob = slice(2 * FEA, 2 * FEA + LAT)
    Wg_s = jnp.concatenate([cat("gate", sl_self), cat("gate", sl_glob)], axis=0)
    Wg_n = cat("gate", sl_nbr)
    b1g = jnp.concatenate([heads[h]["gate"]["layers"][0][1] for h in range(HEADS)])
    W2g = jnp.zeros((HEADS * HID, HEADS), jnp.float32)
    for h in range(HEADS):
        W2g = W2g.at[h * HID:(h + 1) * HID, h].set(heads[h]["gate"]["out"][0][:, 0])
    b2g = jnp.stack([heads[h]["gate"]["out"][1][0] for h in range(HEADS)])
    Wm_s = jnp.concatenate([cat("msg", sl_self), cat("msg", sl_glob)], axis=0)
    Wm_n = cat("msg", sl_nbr)
    b1m = jnp.concatenate([heads[h]["msg"]["layers"][0][1] for h in range(HEADS)])
    W2m = jnp.stack([heads[h]["msg"]["out"][0] for h in range(HEADS)])
    b2m = jnp.stack([heads[h]["msg"]["out"][1].reshape(1, FEA) for h in range(HEADS)])
    return (Wg_s, Wg_n, b1g.reshape(1, -1), W2g, b2g.reshape(1, -1),
            Wm_s, Wm_n, b1m.reshape(1, -1), W2m, b2m)


def kernel(prec_weights, orig_prec_fea, self_fea_idx, nbr_fea_idx, reaction_prec_idx, actions, params):
    pw = prec_weights
    rxn = reaction_prec_idx
    fea, glob, tself = _embed(orig_prec_fea, pw, rxn, params["embed"], actions)

    sidx1 = self_fea_idx.astype(jnp.int32)
    nidx1 = nbr_fea_idx.astype(jnp.int32)
    pw1 = pw.reshape(N)
    zeros128 = jnp.zeros((N, FEA), jnp.float32)
    for heads in params["graphs"]:
        (Wg_s, Wg_n, b1g, W2g, b2g,
         Wm_s, Wm_n, b1m, W2m, b2m) = _pack_layer(heads)
        es, en = _sc_gather(tself, fea, sidx1, nidx1)
        gates = _gates(es, en, Wg_s, Wg_n, b1g, W2g, b2g)
        partials = _sc_segmax(gates, sidx1)
        gmax1 = _maxmerge(partials)
        w = _sc_weights(gates, gmax1, pw1, sidx1, nidx1)
        wmsg, wden = _msgs(es, en, w, Wm_s, Wm_n, b1m, W2m, b2m)
        onum, oden = _sc_acc(wmsg, wden, sidx1, zeros128)
        fea, tself = _norm(onum, oden, fea, glob)

    x = _cry(fea, pw, rxn, params["cry"])
    return _res_mlp(x, params["res"], params["out"])


# speedup vs baseline: 7.0865x; 1.0115x over previous
import dataclasses
import functools

import jax
import jax.numpy as jnp
from jax.experimental import pallas as pl
from jax.experimental.pallas import tpu as pltpu
from jax.experimental.pallas import tpu_sc as plsc

N = 10000
M = 160000
C = 2000
ORIG = 200
FEA = 128
LAT = 32
HEADS = 3
TGT = 64
HID = 256

BM = 640
BN = 400
NEG = -3.0e38


def _leaky(x):
    return jnp.where(x > 0, x, 0.01 * x)


def _embed_body(orig_ref, pw_ref, rxn_ref, Wemb_ref, act_ref, fea_ref, glob_ref, tself_ref):
    orig = orig_ref[...]
    emb = jnp.dot(orig, Wemb_ref[...], preferred_element_type=jnp.float32)
    col = jax.lax.broadcasted_iota(jnp.int32, (BN, FEA), 1)
    fea = jnp.where(col == FEA - 1, pw_ref[...], emb)
    fea_ref[...] = fea
    rxn = rxn_ref[0, 0, :]
    z = (rxn[:, None] == jax.lax.broadcasted_iota(jnp.int32, (BN, C), 1)).astype(jnp.float32)
    glob = jax.lax.dot_general(z, act_ref[...], (((1,), (0,)), ((), ())),
                               preferred_element_type=jnp.float32)
    glob_ref[...] = glob
    tself_ref[...] = jnp.concatenate(
        [fea, glob, jnp.zeros((BN, 96), jnp.float32)], axis=1)


def _embed(orig, pw, rxn, Wemb, actions):
    Wpad = jnp.concatenate([Wemb, jnp.zeros((ORIG, 1), jnp.float32)], axis=1)
    rxn3 = rxn.reshape(N // BN, 1, BN)
    grid = (N // BN,)
    return pl.pallas_call(
        _embed_body,
        grid=grid,
        in_specs=[
            pl.BlockSpec((BN, ORIG), lambda i: (i, 0)),
            pl.BlockSpec((BN, 1), lambda i: (i, 0)),
            pl.BlockSpec((1, 1, BN), lambda i: (i, 0, 0)),
            pl.BlockSpec((ORIG, FEA), lambda i: (0, 0)),
            pl.BlockSpec((C, LAT), lambda i: (0, 0)),
        ],
        out_specs=[
            pl.BlockSpec((BN, FEA), lambda i: (i, 0)),
            pl.BlockSpec((BN, LAT), lambda i: (i, 0)),
            pl.BlockSpec((BN, 256), lambda i: (i, 0)),
        ],
        out_shape=[
            jax.ShapeDtypeStruct((N, FEA), jnp.float32),
            jax.ShapeDtypeStruct((N, LAT), jnp.float32),
            jax.ShapeDtypeStruct((N, 256), jnp.float32),
        ],
    )(orig, pw, rxn3, Wpad, actions)


def _gates_body(es_ref, en_ref, Ws_ref, Wn_ref, b1_ref, W2_ref, b2_ref, out_ref):
    h = jnp.dot(es_ref[:, :160], Ws_ref[...], preferred_element_type=jnp.float32)
    h += jnp.dot(en_ref[...], Wn_ref[...], preferred_element_type=jnp.float32)
    h = _leaky(h + b1_ref[...])
    g = jnp.dot(h, W2_ref[...], preferred_element_type=jnp.float32) + b2_ref[...]
    out_ref[...] = g.T


def _gates(es, en, Ws, Wn, b1, W2, b2):
    grid = (M // BM,)
    return pl.pallas_call(
        _gates_body,
        grid=grid,
        in_specs=[
            pl.BlockSpec((BM, 256), lambda i: (i, 0)),
            pl.BlockSpec((BM, FEA), lambda i: (i, 0)),
            pl.BlockSpec((160, HEADS * HID), lambda i: (0, 0)),
            pl.BlockSpec((FEA, HEADS * HID), lambda i: (0, 0)),
            pl.BlockSpec((1, HEADS * HID), lambda i: (0, 0)),
            pl.BlockSpec((HEADS * HID, HEADS), lambda i: (0, 0)),
            pl.BlockSpec((1, HEADS), lambda i: (0, 0)),
        ],
        out_specs=pl.BlockSpec((HEADS, BM), lambda i: (0, i)),
        out_shape=jax.ShapeDtypeStruct((HEADS, M), jnp.float32),
    )(es, en, Ws, Wn, b1, W2, b2)


def _msgs_body(es_ref, en_ref, w_ref, Ws_ref, Wn_ref, b1_ref, W2_ref, b2_ref, out_ref, den_ref):
    h = jnp.dot(es_ref[:, :160], Ws_ref[...], preferred_element_type=jnp.float32)
    h += jnp.dot(en_ref[...], Wn_ref[...], preferred_element_type=jnp.float32)
    h = _leaky(h + b1_ref[...])
    dens = []
    for hh in range(HEADS):
        m = jnp.dot(h[:, hh * HID:(hh + 1) * HID], W2_ref[hh],
                    preferred_element_type=jnp.float32) + b2_ref[hh]
        wv = w_ref[hh][:, None]
        out_ref[hh] = m * wv
        dens.append(jnp.broadcast_to(wv, (BM, 16)))
    den_ref[...] = jnp.concatenate(
        dens + [jnp.zeros((BM, 128 - 16 * HEADS), jnp.float32)], axis=1)


def _msgs(es, en, w, Ws, Wn, b1, W2, b2):
    grid = (M // BM,)
    return pl.pallas_call(
        _msgs_body,
        grid=grid,
        in_specs=[
            pl.BlockSpec((BM, 256), lambda i: (i, 0)),
            pl.BlockSpec((BM, FEA), lambda i: (i, 0)),
            pl.BlockSpec((HEADS, BM), lambda i: (0, i)),
            pl.BlockSpec((160, HEADS * HID), lambda i: (0, 0)),
            pl.BlockSpec((FEA, HEADS * HID), lambda i: (0, 0)),
            pl.BlockSpec((1, HEADS * HID), lambda i: (0, 0)),
            pl.BlockSpec((HEADS, HID, FEA), lambda i: (0, 0, 0)),
            pl.BlockSpec((HEADS, 1, FEA), lambda i: (0, 0, 0)),
        ],
        out_specs=[
            pl.BlockSpec((HEADS, BM, FEA), lambda i: (0, i, 0)),
            pl.BlockSpec((BM, FEA), lambda i: (i, 0)),
        ],
        out_shape=[
            jax.ShapeDtypeStruct((HEADS, M, FEA), jnp.float32),
            jax.ShapeDtypeStruct((M, FEA), jnp.float32),
        ],
    )(es, en, w, Ws, Wn, b1, W2, b2)


def _norm_body(num_ref, den_ref, fea_ref, glob_ref, fea_o, tself_o):
    av = num_ref[0] + num_ref[1]
    denp = den_ref[0] + den_ref[1]
    acc = jnp.zeros((BN, FEA), jnp.float32)
    for hh in range(HEADS):
        acc += av[hh] / (denp[:, 16 * hh:16 * hh + 1] + 1e-13)
    fea = acc * (1.0 / HEADS) + fea_ref[...]
    fea_o[...] = fea
    tself_o[...] = jnp.concatenate(
        [fea, glob_ref[...], jnp.zeros((BN, 96), jnp.float32)], axis=1)


def _norm(onum, oden, fea, glob):
    grid = (N // BN,)
    return pl.pallas_call(
        _norm_body,
        grid=grid,
        in_specs=[
            pl.BlockSpec((2, HEADS, BN, FEA), lambda i: (0, 0, i, 0)),
            pl.BlockSpec((2, BN, FEA), lambda i: (0, i, 0)),
            pl.BlockSpec((BN, FEA), lambda i: (i, 0)),
            pl.BlockSpec((BN, LAT), lambda i: (i, 0)),
        ],
        out_specs=[
            pl.BlockSpec((BN, FEA), lambda i: (i, 0)),
            pl.BlockSpec((BN, 256), lambda i: (i, 0)),
        ],
        out_shape=[
            jax.ShapeDtypeStruct((N, FEA), jnp.float32),
            jax.ShapeDtypeStruct((N, 256), jnp.float32),
        ],
    )(onum, oden, fea, glob)


def _cry_a_body(fea_ref, rxn_ref, Ws_ref, b1_ref, W2_ref, b2_ref,
                gates_o, gmax_o, gmax_acc):
    i = pl.program_id(0)

    @pl.when(i == 0)
    def _():
        gmax_acc[...] = jnp.full((HEADS, C), NEG, jnp.float32)

    h = _leaky(jnp.dot(fea_ref[...], Ws_ref[...], preferred_element_type=jnp.float32)
               + b1_ref[...])
    g = jnp.dot(h, W2_ref[...], preferred_element_type=jnp.float32) + b2_ref[...]
    gates_o[...] = g
    rxn = rxn_ref[0, 0, :]
    z = rxn[:, None] == jax.lax.broadcasted_iota(jnp.int32, (BN, C), 1)
    for hh in range(HEADS):
        masked = jnp.where(z, g[:, hh:hh + 1], NEG)
        gmax_acc[hh, :] = jnp.maximum(gmax_acc[hh, :], jnp.max(masked, axis=0))

    @pl.when(i == pl.num_programs(0) - 1)
    def _():
        gmax_o[...] = gmax_acc[...]


def _cry_b_body(fea_ref, pw_ref, rxn_ref, gates_ref, gmax_ref,
                Ws_ref, b1_ref, W2_ref, b2_ref, x_o, acc):
    i = pl.program_id(0)

    @pl.when(i == 0)
    def _():
        acc[...] = jnp.zeros((HEADS, C, FEA + 16), jnp.float32)

    h = _leaky(jnp.dot(fea_ref[...], Ws_ref[...], preferred_element_type=jnp.float32)
               + b1_ref[...])
    rxn = rxn_ref[0, 0, :]
    z = rxn[:, None] == jax.lax.broadcasted_iota(jnp.int32, (BN, C), 1)
    zf = z.astype(jnp.float32)
    pw = pw_ref[...]
    for hh in range(HEADS):
        gm = jnp.max(jnp.where(z, gmax_ref[hh, :][None, :], NEG), axis=1)
        e = pw[:, 0] * jnp.exp(gates_ref[:, hh] - gm)
        m = jnp.dot(h[:, hh * HID:(hh + 1) * HID], W2_ref[hh],
                    preferred_element_type=jnp.float32) + b2_ref[hh]
        val = jnp.concatenate(
            [m * e[:, None], e[:, None], jnp.zeros((BN, 15), jnp.float32)], axis=1)
        acc[hh] += jax.lax.dot_general(zf, val, (((0,), (0,)), ((), ())),
                                       preferred_element_type=jnp.float32)

    @pl.when(i == pl.num_programs(0) - 1)
    def _():
        x = jnp.zeros((C, FEA), jnp.float32)
        for hh in range(HEADS):
            a = acc[hh]
            x += a[:, :FEA] / (a[:, FEA:FEA + 1] + 1e-13)
        x_o[...] = x * (1.0 / HEADS)


def _cry(fea, pw, rxn, cry_params):
    Ws = jnp.concatenate([cry_params[h]["gate"]["layers"][0][0] for h in range(HEADS)], axis=1)
    b1 = jnp.concatenate([cry_params[h]["gate"]["layers"][0][1] for h in range(HEADS)])
    W2 = jnp.zeros((HEADS * HID, HEADS), jnp.float32)
    for h in range(HEADS):
        W2 = W2.at[h * HID:(h + 1) * HID, h].set(cry_params[h]["gate"]["out"][0][:, 0])
    b2 = jnp.stack([cry_params[h]["gate"]["out"][1][0] for h in range(HEADS)])
    Wsm = jnp.concatenate([cry_params[h]["msg"]["layers"][0][0] for h in range(HEADS)], axis=1)
    b1m = jnp.concatenate([cry_params[h]["msg"]["layers"][0][1] for h in range(HEADS)])
    W2m = jnp.stack([cry_params[h]["msg"]["out"][0] for h in range(HEADS)])
    b2m = jnp.stack([cry_params[h]["msg"]["out"][1].reshape(1, FEA) for h in range(HEADS)])

    rxn3 = rxn.reshape(N // BN, 1, BN)
    grid = (N // BN,)
    gates, gmax = pl.pallas_call(
        _cry_a_body,
        grid=grid,
        in_specs=[
            pl.BlockSpec((BN, FEA), lambda i: (i, 0)),
            pl.BlockSpec((1, 1, BN), lambda i: (i, 0, 0)),
            pl.BlockSpec((FEA, HEADS * HID), lambda i: (0, 0)),
            pl.BlockSpec((1, HEADS * HID), lambda i: (0, 0)),
            pl.BlockSpec((HEADS * HID, HEADS), lambda i: (0, 0)),
            pl.BlockSpec((1, HEADS), lambda i: (0, 0)),
        ],
        out_specs=[
            pl.BlockSpec((BN, HEADS), lambda i: (i, 0)),
            pl.BlockSpec((HEADS, C), lambda i: (0, 0)),
        ],
        out_shape=[
            jax.ShapeDtypeStruct((N, HEADS), jnp.float32),
            jax.ShapeDtypeStruct((HEADS, C), jnp.float32),
        ],
        scratch_shapes=[pltpu.VMEM((HEADS, C), jnp.float32)],
    )(fea, rxn3, Ws, b1.reshape(1, -1), W2, b2.reshape(1, -1))

    x = pl.pallas_call(
        _cry_b_body,
        grid=grid,
        in_specs=[
            pl.BlockSpec((BN, FEA), lambda i: (i, 0)),
            pl.BlockSpec((BN, 1), lambda i: (i, 0)),
            pl.BlockSpec((1, 1, BN), lambda i: (i, 0, 0)),
            pl.BlockSpec((BN, HEADS), lambda i: (i, 0)),
            pl.BlockSpec((HEADS, C), lambda i: (0, 0)),
            pl.BlockSpec((FEA, HEADS * HID), lambda i: (0, 0)),
            pl.BlockSpec((1, HEADS * HID), lambda i: (0, 0)),
            pl.BlockSpec((HEADS, HID, FEA), lambda i: (0, 0, 0)),
            pl.BlockSpec((HEADS, 1, FEA), lambda i: (0, 0, 0)),
        ],
        out_specs=pl.BlockSpec((C, FEA), lambda i: (0, 0)),
        out_shape=jax.ShapeDtypeStruct((C, FEA), jnp.float32),
        scratch_shapes=[pltpu.VMEM((HEADS, C, FEA + 16), jnp.float32)],
    )(fea, pw, rxn3, gates, gmax, Wsm, b1m.reshape(1, -1), W2m, b2m)
    return x


def _res_mlp_kernel(x_ref, *refs):
    out_ref = refs[-1]
    ws = refs[:-1]
    x = x_ref[...]
    i = 0
    for li, has_res in enumerate((True, True, False, True)):
        W = ws[i][...]
        b = ws[i + 1][...]
        i += 2
        if has_res:
            Wres = ws[i][...]
            i += 1
            res = jnp.dot(x, Wres, preferred_element_type=jnp.float32)
        else:
            res = x
        x = jax.nn.relu(jnp.dot(x, W, preferred_element_type=jnp.float32) + b) + res
    Wo = ws[i][...]
    bo = ws[i + 1][...]
    out_ref[...] = jnp.dot(x, Wo, preferred_element_type=jnp.float32) + bo


def _res_mlp(x, res_layers, out_wb):
    CP = 2048
    xp = jnp.zeros((CP, x.shape[1]), jnp.float32).at[: x.shape[0]].set(x)
    args = []
    for (W, b, Wres) in res_layers:
        args.append(W)
        args.append(b.reshape(1, -1))
        if Wres is not None:
            args.append(Wres)
    Wo, bo = out_wb
    args.append(Wo)
    args.append(bo.reshape(1, -1))
    out = pl.pallas_call(
        _res_mlp_kernel,
        out_shape=jax.ShapeDtypeStruct((CP, TGT), jnp.float32),
    )(xp, *args)
    return out[: x.shape[0]]


_SC_MESH = plsc.VectorSubcoreMesh(core_axis_name="c", subcore_axis_name="s")
_SC_CP = pltpu.CompilerParams()
if "needs_layout_passes" in pltpu.CompilerParams.__dataclass_fields__:
    _SC_CP = dataclasses.replace(_SC_CP, needs_layout_passes=False)
NWORK = 32
GCH = 128
NCHG = M // GCH
CH2 = 640
NCH2 = M // CH2
ACH = 128
NCH3 = M // 2 // ACH


def _sc_gather_body(tself_hbm, fea_hbm, sidx_hbm, nidx_hbm, es_hbm, en_hbm,
                    ibs_a, ibn_a, ibs_b, ibn_b, es_a, en_a, es_b, en_b, sems):
    wid = jax.lax.axis_index("s") * 2 + jax.lax.axis_index("c")
    n = (NCHG - 1 - wid) // NWORK + 1
    nn = (n + 1) // 2

    def base(j):
        return (wid + NWORK * j) * GCH

    def load_idx(j, ibs, ibn):
        pltpu.sync_copy(sidx_hbm.at[pl.ds(base(j), GCH)], ibs)
        pltpu.sync_copy(nidx_hbm.at[pl.ds(base(j), GCH)], ibn)

    def start_gathers(ibs, ibn, esb, enb, semg):
        pltpu.async_copy(tself_hbm.at[ibs], esb, semg)
        pltpu.async_copy(fea_hbm.at[ibn], enb, semg)

    def wait_gathers(esb, enb, semg):
        pltpu.make_async_copy(tself_hbm.at[pl.ds(0, GCH)], esb, semg).wait()
        pltpu.make_async_copy(fea_hbm.at[pl.ds(0, GCH)], enb, semg).wait()

    def start_writes(j, esb, enb, semw):
        pltpu.async_copy(esb, es_hbm.at[pl.ds(base(j), GCH)], semw)
        pltpu.async_copy(enb, en_hbm.at[pl.ds(base(j), GCH)], semw)

    def wait_writes(esb, enb, semw):
        pltpu.make_async_copy(esb, es_hbm.at[pl.ds(0, GCH)], semw).wait()
        pltpu.make_async_copy(enb, en_hbm.at[pl.ds(0, GCH)], semw).wait()

    load_idx(0, ibs_a, ibn_a)
    start_gathers(ibs_a, ibn_a, es_a, en_a, sems.at[0])

    @pl.loop(0, nn)
    def _(jj):
        ja = 2 * jj
        jb = 2 * jj + 1

        @pl.when(jb < n)
        def _():
            @pl.when(jj > 0)
            def _():
                wait_writes(es_b, en_b, sems.at[3])

            load_idx(jb, ibs_b, ibn_b)
            start_gathers(ibs_b, ibn_b, es_b, en_b, sems.at[1])

        wait_gathers(es_a, en_a, sems.at[0])
        start_writes(ja, es_a, en_a, sems.at[2])

        @pl.when(jb < n)
        def _():
            wait_gathers(es_b, en_b, sems.at[1])
            start_writes(jb, es_b, en_b, sems.at[3])

        @pl.when(ja + 2 < n)
        def _():
            wait_writes(es_a, en_a, sems.at[2])
            load_idx(ja + 2, ibs_a, ibn_a)
            start_gathers(ibs_a, ibn_a, es_a, en_a, sems.at[0])

    wait_writes(es_a, en_a, sems.at[2])
    wait_writes(es_b, en_b, sems.at[3])


def _sc_gather(tself, fea, sidx2, nidx2):
    f = pl.kernel(
        _sc_gather_body,
        out_type=(jax.ShapeDtypeStruct((M, 256), jnp.float32),
                  jax.ShapeDtypeStruct((M, FEA), jnp.float32)),
        mesh=_SC_MESH,
        scratch_types=[
            pltpu.VMEM((GCH,), jnp.int32),
            pltpu.VMEM((GCH,), jnp.int32),
            pltpu.VMEM((GCH,), jnp.int32),
            pltpu.VMEM((GCH,), jnp.int32),
            pltpu.VMEM((GCH, 256), jnp.float32),
            pltpu.VMEM((GCH, FEA), jnp.float32),
            pltpu.VMEM((GCH, 256), jnp.float32),
            pltpu.VMEM((GCH, FEA), jnp.float32),
            pltpu.SemaphoreType.DMA((4,)),
        ],
    )
    return f(tself, fea, sidx2, nidx2)


def _sc_segmax_body(gates_hbm, sidx_hbm, out_hbm, acc, ib, gb, sems):
    wid = jax.lax.axis_index("s") * 2 + jax.lax.axis_index("c")

    @pl.loop(0, HEADS * N, step=16)
    def _(i):
        acc[pl.ds(i, 16)] = jnp.full((16,), NEG, jnp.float32)

    @pl.loop(wid, NCH2, step=NWORK)
    def _(k):
        base = k * CH2
        d1 = pltpu.async_copy(sidx_hbm.at[pl.ds(base, CH2)], ib, sems.at[0])
        d2 = pltpu.async_copy(gates_hbm.at[:, pl.ds(base, CH2)], gb, sems.at[1])
        d1.wait()
        d2.wait()

        @pl.loop(0, CH2 // 16)
        def _(v):
            idx = ib[pl.ds(v * 16, 16)]
            for h in range(HEADS):
                g = gb[h, pl.ds(v * 16, 16)]
                idx3 = idx + h * N

                @pl.loop(0, 16)
                def _(r):
                    cur = plsc.load_gather(acc, [idx3])
                    plsc.store_scatter(acc, [idx3], g, mask=g > cur)

    pltpu.sync_copy(acc, out_hbm.at[wid, 0])


def _sc_segmax(gates, sidx1):
    f = pl.kernel(
        _sc_segmax_body,
        out_type=jax.ShapeDtypeStruct((NWORK, 1, HEADS * N), jnp.float32),
        mesh=_SC_MESH,
        scratch_types=[
            pltpu.VMEM((HEADS * N,), jnp.float32),
            pltpu.VMEM((CH2,), jnp.int32),
            pltpu.VMEM((HEADS, CH2), jnp.float32),
            pltpu.SemaphoreType.DMA((2,)),
        ],
        compiler_params=_SC_CP,
    )
    return f(gates, sidx1)


def _maxmerge_body(p_ref, o_ref):
    o_ref[...] = jnp.max(p_ref[:, 0, :], axis=0, keepdims=True)


def _maxmerge(partials):
    out = pl.pallas_call(
        _maxmerge_body,
        out_shape=jax.ShapeDtypeStruct((1, HEADS * N), jnp.float32),
    )(partials)
    return out.reshape(HEADS * N)


def _sc_weights_body(gates_hbm, gmax_hbm, pw_hbm, sidx_hbm, nidx_hbm, w_hbm,
                     gmb, pwb, ibs, ibn, gb, wb, sems):
    wid = jax.lax.axis_index("s") * 2 + jax.lax.axis_index("c")
    pltpu.sync_copy(gmax_hbm, gmb)
    pltpu.sync_copy(pw_hbm, pwb)

    @pl.loop(wid, NCH2, step=NWORK)
    def _(k):
        base = k * CH2
        d1 = pltpu.async_copy(sidx_hbm.at[pl.ds(base, CH2)], ibs, sems.at[0])
        d2 = pltpu.async_copy(nidx_hbm.at[pl.ds(base, CH2)], ibn, sems.at[1])
        d3 = pltpu.async_copy(gates_hbm.at[:, pl.ds(base, CH2)], gb, sems.at[2])
        d1.wait()
        d2.wait()
        d3.wait()

        @pl.loop(0, CH2 // 16)
        def _(v):
            s16 = ibs[pl.ds(v * 16, 16)]
            n16 = ibn[pl.ds(v * 16, 16)]
            pwv = plsc.load_gather(pwb, [n16])
            for h in range(HEADS):
                g = gb[h, pl.ds(v * 16, 16)]
                m = plsc.load_gather(gmb, [s16 + h * N])
                wb[h, pl.ds(v * 16, 16)] = pwv * jnp.exp(g - m)

        d4 = pltpu.async_copy(wb, w_hbm.at[:, pl.ds(base, CH2)], sems.at[3])
        d4.wait()


def _sc_weights(gates, gmax1, pw1, sidx1, nidx1):
    f = pl.kernel(
        _sc_weights_body,
        out_type=jax.ShapeDtypeStruct((HEADS, M), jnp.float32),
        mesh=_SC_MESH,
        scratch_types=[
            pltpu.VMEM((HEADS * N,), jnp.float32),
            pltpu.VMEM((N,), jnp.float32),
            pltpu.VMEM((CH2,), jnp.int32),
            pltpu.VMEM((CH2,), jnp.int32),
            pltpu.VMEM((HEADS, CH2), jnp.float32),
            pltpu.VMEM((HEADS, CH2), jnp.float32),
            pltpu.SemaphoreType.DMA((4,)),
        ],
        compiler_params=_SC_CP,
    )
    return f(gates, gmax1, pw1, sidx1, nidx1)


def _sc_acc_body(wmsg_hbm, wden_hbm, sidx_hbm, zeros_hbm, onum_hbm, oden_hbm,
                 accs, iba, ibb, rba, rbb, sems):
    c = jax.lax.axis_index("c")
    s = jax.lax.axis_index("s")
    zoff = s * 640

    n = (NCH3 - 1 - s) // 16 + 1
    nn = (n + 1) // 2

    for h in range(HEADS + 1):
        if h < HEADS:
            src_all = wmsg_hbm.at[h]
        else:
            src_all = wden_hbm

        @pl.when(s < 15)
        def _():
            pltpu.sync_copy(zeros_hbm.at[pl.ds(zoff, 640)], accs.at[pl.ds(zoff, 640)])

        @pl.when(s == 15)
        def _():
            pltpu.sync_copy(zeros_hbm.at[pl.ds(9600, 400)], accs.at[pl.ds(9600, 400)])

        plsc.subcore_barrier()

        def issue(j, ib, rb, sem):
            row = c * NCH3 + s + 16 * j
            pltpu.async_copy(sidx_hbm.at[pl.ds(row * ACH, ACH)], ib, sem)
            pltpu.async_copy(src_all.at[pl.ds(row * ACH, ACH)], rb, sem)

        def wait_and_scatter(ib, rb, sem):
            pltpu.make_async_copy(sidx_hbm.at[pl.ds(0, ACH)], ib, sem).wait()
            pltpu.make_async_copy(src_all.at[pl.ds(0, ACH)], rb, sem).wait()
            pltpu.sync_copy(rb, accs.at[ib], add=True)

        issue(0, iba, rba, sems.at[0])

        @pl.loop(0, nn)
        def _(jj):
            ja = 2 * jj
            jb = 2 * jj + 1

            @pl.when(jb < n)
            def _():
                issue(jb, ibb, rbb, sems.at[1])

            wait_and_scatter(iba, rba, sems.at[0])

            @pl.when(ja + 2 < n)
            def _():
                issue(ja + 2, iba, rba, sems.at[0])

            @pl.when(jb < n)
            def _():
                wait_and_scatter(ibb, rbb, sems.at[1])

        plsc.subcore_barrier()

        if h < HEADS:
            dst_all = onum_hbm.at[c, h]
        else:
            dst_all = oden_hbm.at[c]

        @pl.when(s < 15)
        def _():
            pltpu.sync_copy(accs.at[pl.ds(zoff, 640)], dst_all.at[pl.ds(zoff, 640)])

        @pl.when(s == 15)
        def _():
            pltpu.sync_copy(accs.at[pl.ds(9600, 400)], dst_all.at[pl.ds(9600, 400)])

        plsc.subcore_barrier()


def _sc_acc(wmsg, wden, sidx1, zeros128):
    f = pl.kernel(
        _sc_acc_body,
        out_type=(jax.ShapeDtypeStruct((2, HEADS, N, FEA), jnp.float32),
                  jax.ShapeDtypeStruct((2, N, FEA), jnp.float32)),
        mesh=_SC_MESH,
        scratch_types=[
            pltpu.VMEM_SHARED((N, FEA), jnp.float32),
            pltpu.VMEM((128,), jnp.int32),
            pltpu.VMEM((128,), jnp.int32),
            pltpu.VMEM((ACH, FEA), jnp.float32),
            pltpu.VMEM((ACH, FEA), jnp.float32),
            pltpu.SemaphoreType.DMA((2,)),
        ],
    )
    return f(wmsg, wden, sidx1, zeros128)


def _pack_layer(heads):
    def cat(key, part):
        return jnp.concatenate([heads[h][key]["layers"][0][0][part] for h in range(HEADS)], axis=1)
    sl_self = slice(0, FEA)
    sl_nbr = slice(FEA, 2 * FEA)
    sl_glob = slice(2 * FEA, 2 * FEA + LAT)
    Wg_s = jnp.concatenate([cat("gate", sl_self), cat("gate", sl_glob)], axis=0)
    Wg_n = cat("gate", sl_nbr)
    b1g = jnp.concatenate([heads[h]["gate"]["layers"][0][1] for h in range(HEADS)])
    W2g = jnp.zeros((HEADS * HID, HEADS), jnp.float32)
    for h in range(HEADS):
        W2g = W2g.at[h * HID:(h + 1) * HID, h].set(heads[h]["gate"]["out"][0][:, 0])
    b2g = jnp.stack([heads[h]["gate"]["out"][1][0] for h in range(HEADS)])
    Wm_s = jnp.concatenate([cat("msg", sl_self), cat("msg", sl_glob)], axis=0)
    Wm_n = cat("msg", sl_nbr)
    b1m = jnp.concatenate([heads[h]["msg"]["layers"][0][1] for h in range(HEADS)])
    W2m = jnp.stack([heads[h]["msg"]["out"][0] for h in range(HEADS)])
    b2m = jnp.stack([heads[h]["msg"]["out"][1].reshape(1, FEA) for h in range(HEADS)])
    return (Wg_s, Wg_n, b1g.reshape(1, -1), W2g, b2g.reshape(1, -1),
            Wm_s, Wm_n, b1m.reshape(1, -1), W2m, b2m)


def kernel(prec_weights, orig_prec_fea, self_fea_idx, nbr_fea_idx, reaction_prec_idx, actions, params):
    pw = prec_weights
    rxn = reaction_prec_idx
    fea, glob, tself = _embed(orig_prec_fea, pw, rxn, params["embed"], actions)

    sidx1 = self_fea_idx.astype(jnp.int32)
    nidx1 = nbr_fea_idx.astype(jnp.int32)
    pw1 = pw.reshape(N)
    zeros128 = jnp.zeros((N, FEA), jnp.float32)
    for heads in params["graphs"]:
        (Wg_s, Wg_n, b1g, W2g, b2g,
         Wm_s, Wm_n, b1m, W2m, b2m) = _pack_layer(heads)
        es, en = _sc_gather(tself, fea, sidx1, nidx1)
        gates = _gates(es, en, Wg_s, Wg_n, b1g, W2g, b2g)
        partials = _sc_segmax(gates, sidx1)
        gmax1 = _maxmerge(partials)
        w = _sc_weights(gates, gmax1, pw1, sidx1, nidx1)
        wmsg, wden = _msgs(es, en, w, Wm_s, Wm_n, b1m, W2m, b2m)
        onum, oden = _sc_acc(wmsg, wden, sidx1, zeros128)
        fea, tself = _norm(onum, oden, fea, glob)

    x = _cry(fea, pw, rxn, params["cry"])
    return _res_mlp(x, params["res"], params["out"])


# speedup vs baseline: 7.1151x; 1.0040x over previous
import dataclasses
import functools

import jax
import jax.numpy as jnp
from jax.experimental import pallas as pl
from jax.experimental.pallas import tpu as pltpu
from jax.experimental.pallas import tpu_sc as plsc

N = 10000
M = 160000
C = 2000
ORIG = 200
FEA = 128
LAT = 32
HEADS = 3
TGT = 64
HID = 256

BM = 640
BN = 400
NEG = -3.0e38


def _leaky(x):
    return jnp.where(x > 0, x, 0.01 * x)


def _embed_body(orig_ref, pw_ref, rxn_ref, Wemb_ref, act_ref, fea_ref, glob_ref, tself_ref):
    orig = orig_ref[...]
    emb = jnp.dot(orig, Wemb_ref[...], preferred_element_type=jnp.float32)
    col = jax.lax.broadcasted_iota(jnp.int32, (BN, FEA), 1)
    fea = jnp.where(col == FEA - 1, pw_ref[...], emb)
    fea_ref[...] = fea
    rxn = rxn_ref[0, 0, :]
    z = (rxn[:, None] == jax.lax.broadcasted_iota(jnp.int32, (BN, C), 1)).astype(jnp.float32)
    glob = jax.lax.dot_general(z, act_ref[...], (((1,), (0,)), ((), ())),
                               preferred_element_type=jnp.float32)
    glob_ref[...] = glob
    tself_ref[...] = jnp.concatenate(
        [fea, glob, jnp.zeros((BN, 96), jnp.float32)], axis=1)


def _embed(orig, pw, rxn, Wemb, actions):
    Wpad = jnp.concatenate([Wemb, jnp.zeros((ORIG, 1), jnp.float32)], axis=1)
    rxn3 = rxn.reshape(N // BN, 1, BN)
    grid = (N // BN,)
    return pl.pallas_call(
        _embed_body,
        grid=grid,
        in_specs=[
            pl.BlockSpec((BN, ORIG), lambda i: (i, 0)),
            pl.BlockSpec((BN, 1), lambda i: (i, 0)),
            pl.BlockSpec((1, 1, BN), lambda i: (i, 0, 0)),
            pl.BlockSpec((ORIG, FEA), lambda i: (0, 0)),
            pl.BlockSpec((C, LAT), lambda i: (0, 0)),
        ],
        out_specs=[
            pl.BlockSpec((BN, FEA), lambda i: (i, 0)),
            pl.BlockSpec((BN, LAT), lambda i: (i, 0)),
            pl.BlockSpec((BN, 256), lambda i: (i, 0)),
        ],
        out_shape=[
            jax.ShapeDtypeStruct((N, FEA), jnp.float32),
            jax.ShapeDtypeStruct((N, LAT), jnp.float32),
            jax.ShapeDtypeStruct((N, 256), jnp.float32),
        ],
    )(orig, pw, rxn3, Wpad, actions)


def _gates_body(es_ref, en_ref, Ws_ref, Wn_ref, b1_ref, W2_ref, b2_ref, out_ref):
    bf = jnp.bfloat16
    h = jnp.dot(es_ref[:, :160].astype(bf), Ws_ref[...].astype(bf),
                preferred_element_type=jnp.float32)
    h += jnp.dot(en_ref[...].astype(bf), Wn_ref[...].astype(bf),
                 preferred_element_type=jnp.float32)
    h = _leaky(h + b1_ref[...])
    g = jnp.dot(h, W2_ref[...], preferred_element_type=jnp.float32) + b2_ref[...]
    out_ref[...] = g.T


def _gates(es, en, Ws, Wn, b1, W2, b2):
    grid = (M // BM,)
    return pl.pallas_call(
        _gates_body,
        grid=grid,
        in_specs=[
            pl.BlockSpec((BM, 256), lambda i: (i, 0)),
            pl.BlockSpec((BM, FEA), lambda i: (i, 0)),
            pl.BlockSpec((160, HEADS * HID), lambda i: (0, 0)),
            pl.BlockSpec((FEA, HEADS * HID), lambda i: (0, 0)),
            pl.BlockSpec((1, HEADS * HID), lambda i: (0, 0)),
            pl.BlockSpec((HEADS * HID, HEADS), lambda i: (0, 0)),
            pl.BlockSpec((1, HEADS), lambda i: (0, 0)),
        ],
        out_specs=pl.BlockSpec((HEADS, BM), lambda i: (0, i)),
        out_shape=jax.ShapeDtypeStruct((HEADS, M), jnp.float32),
    )(es, en, Ws, Wn, b1, W2, b2)


def _msgs_body(es_ref, en_ref, w_ref, Ws_ref, Wn_ref, b1_ref, W2_ref, b2_ref, out_ref, den_ref):
    bf = jnp.bfloat16
    h = jnp.dot(es_ref[:, :160].astype(bf), Ws_ref[...].astype(bf),
                preferred_element_type=jnp.float32)
    h += jnp.dot(en_ref[...].astype(bf), Wn_ref[...].astype(bf),
                 preferred_element_type=jnp.float32)
    h = _leaky(h + b1_ref[...])
    dens = []
    for hh in range(HEADS):
        m = jnp.dot(h[:, hh * HID:(hh + 1) * HID], W2_ref[hh],
                    preferred_element_type=jnp.float32) + b2_ref[hh]
        wv = w_ref[hh][:, None]
        out_ref[hh] = m * wv
        dens.append(jnp.broadcast_to(wv, (BM, 16)))
    den_ref[...] = jnp.concatenate(
        dens + [jnp.zeros((BM, 128 - 16 * HEADS), jnp.float32)], axis=1)


def _msgs(es, en, w, Ws, Wn, b1, W2, b2):
    grid = (M // BM,)
    return pl.pallas_call(
        _msgs_body,
        grid=grid,
        in_specs=[
            pl.BlockSpec((BM, 256), lambda i: (i, 0)),
            pl.BlockSpec((BM, FEA), lambda i: (i, 0)),
            pl.BlockSpec((HEADS, BM), lambda i: (0, i)),
            pl.BlockSpec((160, HEADS * HID), lambda i: (0, 0)),
            pl.BlockSpec((FEA, HEADS * HID), lambda i: (0, 0)),
            pl.BlockSpec((1, HEADS * HID), lambda i: (0, 0)),
            pl.BlockSpec((HEADS, HID, FEA), lambda i: (0, 0, 0)),
            pl.BlockSpec((HEADS, 1, FEA), lambda i: (0, 0, 0)),
        ],
        out_specs=[
            pl.BlockSpec((HEADS, BM, FEA), lambda i: (0, i, 0)),
            pl.BlockSpec((BM, FEA), lambda i: (i, 0)),
        ],
        out_shape=[
            jax.ShapeDtypeStruct((HEADS, M, FEA), jnp.float32),
            jax.ShapeDtypeStruct((M, FEA), jnp.float32),
        ],
    )(es, en, w, Ws, Wn, b1, W2, b2)


def _norm_body(num_ref, den_ref, fea_ref, glob_ref, fea_o, tself_o):
    av = num_ref[0] + num_ref[1]
    denp = den_ref[0] + den_ref[1]
    acc = jnp.zeros((BN, FEA), jnp.float32)
    for hh in range(HEADS):
        acc += av[hh] / (denp[:, 16 * hh:16 * hh + 1] + 1e-13)
    fea = acc * (1.0 / HEADS) + fea_ref[...]
    fea_o[...] = fea
    tself_o[...] = jnp.concatenate(
        [fea, glob_ref[...], jnp.zeros((BN, 96), jnp.float32)], axis=1)


def _norm(onum, oden, fea, glob):
    grid = (N // BN,)
    return pl.pallas_call(
        _norm_body,
        grid=grid,
        in_specs=[
            pl.BlockSpec((2, HEADS, BN, FEA), lambda i: (0, 0, i, 0)),
            pl.BlockSpec((2, BN, FEA), lambda i: (0, i, 0)),
            pl.BlockSpec((BN, FEA), lambda i: (i, 0)),
            pl.BlockSpec((BN, LAT), lambda i: (i, 0)),
        ],
        out_specs=[
            pl.BlockSpec((BN, FEA), lambda i: (i, 0)),
            pl.BlockSpec((BN, 256), lambda i: (i, 0)),
        ],
        out_shape=[
            jax.ShapeDtypeStruct((N, FEA), jnp.float32),
            jax.ShapeDtypeStruct((N, 256), jnp.float32),
        ],
    )(onum, oden, fea, glob)


def _cry_a_body(fea_ref, rxn_ref, Ws_ref, b1_ref, W2_ref, b2_ref,
                gates_o, gmax_o, gmax_acc):
    i = pl.program_id(0)

    @pl.when(i == 0)
    def _():
        gmax_acc[...] = jnp.full((HEADS, C), NEG, jnp.float32)

    h = _leaky(jnp.dot(fea_ref[...], Ws_ref[...], preferred_element_type=jnp.float32)
               + b1_ref[...])
    g = jnp.dot(h, W2_ref[...], preferred_element_type=jnp.float32) + b2_ref[...]
    gates_o[...] = g
    rxn = rxn_ref[0, 0, :]
    z = rxn[:, None] == jax.lax.broadcasted_iota(jnp.int32, (BN, C), 1)
    for hh in range(HEADS):
        masked = jnp.where(z, g[:, hh:hh + 1], NEG)
        gmax_acc[hh, :] = jnp.maximum(gmax_acc[hh, :], jnp.max(masked, axis=0))

    @pl.when(i == pl.num_programs(0) - 1)
    def _():
        gmax_o[...] = gmax_acc[...]


def _cry_b_body(fea_ref, pw_ref, rxn_ref, gates_ref, gmax_ref,
                Ws_ref, b1_ref, W2_ref, b2_ref, x_o, acc):
    i = pl.program_id(0)

    @pl.when(i == 0)
    def _():
        acc[...] = jnp.zeros((HEADS, C, FEA + 16), jnp.float32)

    h = _leaky(jnp.dot(fea_ref[...], Ws_ref[...], preferred_element_type=jnp.float32)
               + b1_ref[...])
    rxn = rxn_ref[0, 0, :]
    z = rxn[:, None] == jax.lax.broadcasted_iota(jnp.int32, (BN, C), 1)
    zf = z.astype(jnp.float32)
    pw = pw_ref[...]
    for hh in range(HEADS):
        gm = jnp.max(jnp.where(z, gmax_ref[hh, :][None, :], NEG), axis=1)
        e = pw[:, 0] * jnp.exp(gates_ref[:, hh] - gm)
        m = jnp.dot(h[:, hh * HID:(hh + 1) * HID], W2_ref[hh],
                    preferred_element_type=jnp.float32) + b2_ref[hh]
        val = jnp.concatenate(
            [m * e[:, None], e[:, None], jnp.zeros((BN, 15), jnp.float32)], axis=1)
        acc[hh] += jax.lax.dot_general(zf, val, (((0,), (0,)), ((), ())),
                                       preferred_element_type=jnp.float32)

    @pl.when(i == pl.num_programs(0) - 1)
    def _():
        x = jnp.zeros((C, FEA), jnp.float32)
        for hh in range(HEADS):
            a = acc[hh]
            x += a[:, :FEA] / (a[:, FEA:FEA + 1] + 1e-13)
        x_o[...] = x * (1.0 / HEADS)


def _cry(fea, pw, rxn, cry_params):
    Ws = jnp.concatenate([cry_params[h]["gate"]["layers"][0][0] for h in range(HEADS)], axis=1)
    b1 = jnp.concatenate([cry_params[h]["gate"]["layers"][0][1] for h in range(HEADS)])
    W2 = jnp.zeros((HEADS * HID, HEADS), jnp.float32)
    for h in range(HEADS):
        W2 = W2.at[h * HID:(h + 1) * HID, h].set(cry_params[h]["gate"]["out"][0][:, 0])
    b2 = jnp.stack([cry_params[h]["gate"]["out"][1][0] for h in range(HEADS)])
    Wsm = jnp.concatenate([cry_params[h]["msg"]["layers"][0][0] for h in range(HEADS)], axis=1)
    b1m = jnp.concatenate([cry_params[h]["msg"]["layers"][0][1] for h in range(HEADS)])
    W2m = jnp.stack([cry_params[h]["msg"]["out"][0] for h in range(HEADS)])
    b2m = jnp.stack([cry_params[h]["msg"]["out"][1].reshape(1, FEA) for h in range(HEADS)])

    rxn3 = rxn.reshape(N // BN, 1, BN)
    grid = (N // BN,)
    gates, gmax = pl.pallas_call(
        _cry_a_body,
        grid=grid,
        in_specs=[
            pl.BlockSpec((BN, FEA), lambda i: (i, 0)),
            pl.BlockSpec((1, 1, BN), lambda i: (i, 0, 0)),
            pl.BlockSpec((FEA, HEADS * HID), lambda i: (0, 0)),
            pl.BlockSpec((1, HEADS * HID), lambda i: (0, 0)),
            pl.BlockSpec((HEADS * HID, HEADS), lambda i: (0, 0)),
            pl.BlockSpec((1, HEADS), lambda i: (0, 0)),
        ],
        out_specs=[
            pl.BlockSpec((BN, HEADS), lambda i: (i, 0)),
            pl.BlockSpec((HEADS, C), lambda i: (0, 0)),
        ],
        out_shape=[
            jax.ShapeDtypeStruct((N, HEADS), jnp.float32),
            jax.ShapeDtypeStruct((HEADS, C), jnp.float32),
        ],
        scratch_shapes=[pltpu.VMEM((HEADS, C), jnp.float32)],
    )(fea, rxn3, Ws, b1.reshape(1, -1), W2, b2.reshape(1, -1))

    x = pl.pallas_call(
        _cry_b_body,
        grid=grid,
        in_specs=[
            pl.BlockSpec((BN, FEA), lambda i: (i, 0)),
            pl.BlockSpec((BN, 1), lambda i: (i, 0)),
            pl.BlockSpec((1, 1, BN), lambda i: (i, 0, 0)),
            pl.BlockSpec((BN, HEADS), lambda i: (i, 0)),
            pl.BlockSpec((HEADS, C), lambda i: (0, 0)),
            pl.BlockSpec((FEA, HEADS * HID), lambda i: (0, 0)),
            pl.BlockSpec((1, HEADS * HID), lambda i: (0, 0)),
            pl.BlockSpec((HEADS, HID, FEA), lambda i: (0, 0, 0)),
            pl.BlockSpec((HEADS, 1, FEA), lambda i: (0, 0, 0)),
        ],
        out_specs=pl.BlockSpec((C, FEA), lambda i: (0, 0)),
        out_shape=jax.ShapeDtypeStruct((C, FEA), jnp.float32),
        scratch_shapes=[pltpu.VMEM((HEADS, C, FEA + 16), jnp.float32)],
    )(fea, pw, rxn3, gates, gmax, Wsm, b1m.reshape(1, -1), W2m, b2m)
    return x


def _res_mlp_kernel(x_ref, *refs):
    out_ref = refs[-1]
    ws = refs[:-1]
    x = x_ref[...]
    i = 0
    for li, has_res in enumerate((True, True, False, True)):
        W = ws[i][...]
        b = ws[i + 1][...]
        i += 2
        if has_res:
            Wres = ws[i][...]
            i += 1
            res = jnp.dot(x, Wres, preferred_element_type=jnp.float32)
        else:
            res = x
        x = jax.nn.relu(jnp.dot(x, W, preferred_element_type=jnp.float32) + b) + res
    Wo = ws[i][...]
    bo = ws[i + 1][...]
    out_ref[...] = jnp.dot(x, Wo, preferred_element_type=jnp.float32) + bo


def _res_mlp(x, res_layers, out_wb):
    CP = 2048
    xp = jnp.zeros((CP, x.shape[1]), jnp.float32).at[: x.shape[0]].set(x)
    args = []
    for (W, b, Wres) in res_layers:
        args.append(W)
        args.append(b.reshape(1, -1))
        if Wres is not None:
            args.append(Wres)
    Wo, bo = out_wb
    args.append(Wo)
    args.append(bo.reshape(1, -1))
    out = pl.pallas_call(
        _res_mlp_kernel,
        out_shape=jax.ShapeDtypeStruct((CP, TGT), jnp.float32),
    )(xp, *args)
    return out[: x.shape[0]]


_SC_MESH = plsc.VectorSubcoreMesh(core_axis_name="c", subcore_axis_name="s")
_SC_CP = pltpu.CompilerParams()
if "needs_layout_passes" in pltpu.CompilerParams.__dataclass_fields__:
    _SC_CP = dataclasses.replace(_SC_CP, needs_layout_passes=False)
NWORK = 32
GCH = 128
NCHG = M // GCH
CH2 = 640
NCH2 = M // CH2
ACH = 128
NCH3 = M // 2 // ACH


def _sc_gather_body(tself_hbm, fea_hbm, sidx_hbm, nidx_hbm, es_hbm, en_hbm,
                    ibs_a, ibn_a, ibs_b, ibn_b, es_a, en_a, es_b, en_b, sems):
    wid = jax.lax.axis_index("s") * 2 + jax.lax.axis_index("c")
    n = (NCHG - 1 - wid) // NWORK + 1
    nn = (n + 1) // 2

    def base(j):
        return (wid + NWORK * j) * GCH

    def load_idx(j, ibs, ibn):
        pltpu.sync_copy(sidx_hbm.at[pl.ds(base(j), GCH)], ibs)
        pltpu.sync_copy(nidx_hbm.at[pl.ds(base(j), GCH)], ibn)

    def start_gathers(ibs, ibn, esb, enb, semg):
        pltpu.async_copy(tself_hbm.at[ibs], esb, semg)
        pltpu.async_copy(fea_hbm.at[ibn], enb, semg)

    def wait_gathers(esb, enb, semg):
        pltpu.make_async_copy(tself_hbm.at[pl.ds(0, GCH)], esb, semg).wait()
        pltpu.make_async_copy(fea_hbm.at[pl.ds(0, GCH)], enb, semg).wait()

    def start_writes(j, esb, enb, semw):
        pltpu.async_copy(esb, es_hbm.at[pl.ds(base(j), GCH)], semw)
        pltpu.async_copy(enb, en_hbm.at[pl.ds(base(j), GCH)], semw)

    def wait_writes(esb, enb, semw):
        pltpu.make_async_copy(esb, es_hbm.at[pl.ds(0, GCH)], semw).wait()
        pltpu.make_async_copy(enb, en_hbm.at[pl.ds(0, GCH)], semw).wait()

    load_idx(0, ibs_a, ibn_a)
    start_gathers(ibs_a, ibn_a, es_a, en_a, sems.at[0])

    @pl.loop(0, nn)
    def _(jj):
        ja = 2 * jj
        jb = 2 * jj + 1

        @pl.when(jb < n)
        def _():
            @pl.when(jj > 0)
            def _():
                wait_writes(es_b, en_b, sems.at[3])

            load_idx(jb, ibs_b, ibn_b)
            start_gathers(ibs_b, ibn_b, es_b, en_b, sems.at[1])

        wait_gathers(es_a, en_a, sems.at[0])
        start_writes(ja, es_a, en_a, sems.at[2])

        @pl.when(jb < n)
        def _():
            wait_gathers(es_b, en_b, sems.at[1])
            start_writes(jb, es_b, en_b, sems.at[3])

        @pl.when(ja + 2 < n)
        def _():
            wait_writes(es_a, en_a, sems.at[2])
            load_idx(ja + 2, ibs_a, ibn_a)
            start_gathers(ibs_a, ibn_a, es_a, en_a, sems.at[0])

    wait_writes(es_a, en_a, sems.at[2])
    wait_writes(es_b, en_b, sems.at[3])


def _sc_gather(tself, fea, sidx2, nidx2):
    f = pl.kernel(
        _sc_gather_body,
        out_type=(jax.ShapeDtypeStruct((M, 256), jnp.float32),
                  jax.ShapeDtypeStruct((M, FEA), jnp.float32)),
        mesh=_SC_MESH,
        scratch_types=[
            pltpu.VMEM((GCH,), jnp.int32),
            pltpu.VMEM((GCH,), jnp.int32),
            pltpu.VMEM((GCH,), jnp.int32),
            pltpu.VMEM((GCH,), jnp.int32),
            pltpu.VMEM((GCH, 256), jnp.float32),
            pltpu.VMEM((GCH, FEA), jnp.float32),
            pltpu.VMEM((GCH, 256), jnp.float32),
            pltpu.VMEM((GCH, FEA), jnp.float32),
            pltpu.SemaphoreType.DMA((4,)),
        ],
    )
    return f(tself, fea, sidx2, nidx2)


def _sc_segmax_body(gates_hbm, sidx_hbm, out_hbm, acc, ib, gb, sems):
    wid = jax.lax.axis_index("s") * 2 + jax.lax.axis_index("c")

    @pl.loop(0, HEADS * N, step=16)
    def _(i):
        acc[pl.ds(i, 16)] = jnp.full((16,), NEG, jnp.float32)

    @pl.loop(wid, NCH2, step=NWORK)
    def _(k):
        base = k * CH2
        d1 = pltpu.async_copy(sidx_hbm.at[pl.ds(base, CH2)], ib, sems.at[0])
        d2 = pltpu.async_copy(gates_hbm.at[:, pl.ds(base, CH2)], gb, sems.at[1])
        d1.wait()
        d2.wait()

        @pl.loop(0, CH2 // 16)
        def _(v):
            idx = ib[pl.ds(v * 16, 16)]
            for h in range(HEADS):
                g = gb[h, pl.ds(v * 16, 16)]
                idx3 = idx + h * N

                @pl.loop(0, 16)
                def _(r):
                    cur = plsc.load_gather(acc, [idx3])
                    plsc.store_scatter(acc, [idx3], g, mask=g > cur)

    pltpu.sync_copy(acc, out_hbm.at[wid, 0])


def _sc_segmax(gates, sidx1):
    f = pl.kernel(
        _sc_segmax_body,
        out_type=jax.ShapeDtypeStruct((NWORK, 1, HEADS * N), jnp.float32),
        mesh=_SC_MESH,
        scratch_types=[
            pltpu.VMEM((HEADS * N,), jnp.float32),
            pltpu.VMEM((CH2,), jnp.int32),
            pltpu.VMEM((HEADS, CH2), jnp.float32),
            pltpu.SemaphoreType.DMA((2,)),
        ],
        compiler_params=_SC_CP,
    )
    return f(gates, sidx1)


def _maxmerge_body(p_ref, o_ref):
    o_ref[...] = jnp.max(p_ref[:, 0, :], axis=0, keepdims=True)


def _maxmerge(partials):
    out = pl.pallas_call(
        _maxmerge_body,
        out_shape=jax.ShapeDtypeStruct((1, HEADS * N), jnp.float32),
    )(partials)
    return out.reshape(HEADS * N)


def _sc_weights_body(gates_hbm, gmax_hbm, pw_hbm, sidx_hbm, nidx_hbm, w_hbm,
                     gmb, pwb, ibs, ibn, gb, wb, sems):
    wid = jax.lax.axis_index("s") * 2 + jax.lax.axis_index("c")
    pltpu.sync_copy(gmax_hbm, gmb)
    pltpu.sync_copy(pw_hbm, pwb)

    @pl.loop(wid, NCH2, step=NWORK)
    def _(k):
        base = k * CH2
        d1 = pltpu.async_copy(sidx_hbm.at[pl.ds(base, CH2)], ibs, sems.at[0])
        d2 = pltpu.async_copy(nidx_hbm.at[pl.ds(base, CH2)], ibn, sems.at[1])
        d3 = pltpu.async_copy(gates_hbm.at[:, pl.ds(base, CH2)], gb, sems.at[2])
        d1.wait()
        d2.wait()
        d3.wait()

        @pl.loop(0, CH2 // 16)
        def _(v):
            s16 = ibs[pl.ds(v * 16, 16)]
            n16 = ibn[pl.ds(v * 16, 16)]
            pwv = plsc.load_gather(pwb, [n16])
            for h in range(HEADS):
                g = gb[h, pl.ds(v * 16, 16)]
                m = plsc.load_gather(gmb, [s16 + h * N])
                wb[h, pl.ds(v * 16, 16)] = pwv * jnp.exp(g - m)

        d4 = pltpu.async_copy(wb, w_hbm.at[:, pl.ds(base, CH2)], sems.at[3])
        d4.wait()


def _sc_weights(gates, gmax1, pw1, sidx1, nidx1):
    f = pl.kernel(
        _sc_weights_body,
        out_type=jax.ShapeDtypeStruct((HEADS, M), jnp.float32),
        mesh=_SC_MESH,
        scratch_types=[
            pltpu.VMEM((HEADS * N,), jnp.float32),
            pltpu.VMEM((N,), jnp.float32),
            pltpu.VMEM((CH2,), jnp.int32),
            pltpu.VMEM((CH2,), jnp.int32),
            pltpu.VMEM((HEADS, CH2), jnp.float32),
            pltpu.VMEM((HEADS, CH2), jnp.float32),
            pltpu.SemaphoreType.DMA((4,)),
        ],
        compiler_params=_SC_CP,
    )
    return f(gates, gmax1, pw1, sidx1, nidx1)


def _sc_acc_body(wmsg_hbm, wden_hbm, sidx_hbm, zeros_hbm, onum_hbm, oden_hbm,
                 accs, iba, ibb, rba, rbb, sems):
    c = jax.lax.axis_index("c")
    s = jax.lax.axis_index("s")
    zoff = s * 640

    n = (NCH3 - 1 - s) // 16 + 1
    nn = (n + 1) // 2

    for h in range(HEADS + 1):
        if h < HEADS:
            src_all = wmsg_hbm.at[h]
        else:
            src_all = wden_hbm

        @pl.when(s < 15)
        def _():
            pltpu.sync_copy(zeros_hbm.at[pl.ds(zoff, 640)], accs.at[pl.ds(zoff, 640)])

        @pl.when(s == 15)
        def _():
            pltpu.sync_copy(zeros_hbm.at[pl.ds(9600, 400)], accs.at[pl.ds(9600, 400)])

        plsc.subcore_barrier()

        def issue(j, ib, rb, sem):
            row = c * NCH3 + s + 16 * j
            pltpu.async_copy(sidx_hbm.at[pl.ds(row * ACH, ACH)], ib, sem)
            pltpu.async_copy(src_all.at[pl.ds(row * ACH, ACH)], rb, sem)

        def wait_and_scatter(ib, rb, sem):
            pltpu.make_async_copy(sidx_hbm.at[pl.ds(0, ACH)], ib, sem).wait()
            pltpu.make_async_copy(src_all.at[pl.ds(0, ACH)], rb, sem).wait()
            pltpu.sync_copy(rb, accs.at[ib], add=True)

        issue(0, iba, rba, sems.at[0])

        @pl.loop(0, nn)
        def _(jj):
            ja = 2 * jj
            jb = 2 * jj + 1

            @pl.when(jb < n)
            def _():
                issue(jb, ibb, rbb, sems.at[1])

            wait_and_scatter(iba, rba, sems.at[0])

            @pl.when(ja + 2 < n)
            def _():
                issue(ja + 2, iba, rba, sems.at[0])

            @pl.when(jb < n)
            def _():
                wait_and_scatter(ibb, rbb, sems.at[1])

        plsc.subcore_barrier()

        if h < HEADS:
            dst_all = onum_hbm.at[c, h]
        else:
            dst_all = oden_hbm.at[c]

        @pl.when(s < 15)
        def _():
            pltpu.sync_copy(accs.at[pl.ds(zoff, 640)], dst_all.at[pl.ds(zoff, 640)])

        @pl.when(s == 15)
        def _():
            pltpu.sync_copy(accs.at[pl.ds(9600, 400)], dst_all.at[pl.ds(9600, 400)])

        plsc.subcore_barrier()


def _sc_acc(wmsg, wden, sidx1, zeros128):
    f = pl.kernel(
        _sc_acc_body,
        out_type=(jax.ShapeDtypeStruct((2, HEADS, N, FEA), jnp.float32),
                  jax.ShapeDtypeStruct((2, N, FEA), jnp.float32)),
        mesh=_SC_MESH,
        scratch_types=[
            pltpu.VMEM_SHARED((N, FEA), jnp.float32),
            pltpu.VMEM((128,), jnp.int32),
            pltpu.VMEM((128,), jnp.int32),
            pltpu.VMEM((ACH, FEA), jnp.float32),
            pltpu.VMEM((ACH, FEA), jnp.float32),
            pltpu.SemaphoreType.DMA((2,)),
        ],
    )
    return f(wmsg, wden, sidx1, zeros128)


def _pack_layer(heads):
    def cat(key, part):
        return jnp.concatenate([heads[h][key]["layers"][0][0][part] for h in range(HEADS)], axis=1)
    sl_self = slice(0, FEA)
    sl_nbr = slice(FEA, 2 * FEA)
    sl_glob = slice(2 * FEA, 2 * FEA + LAT)
    Wg_s = jnp.concatenate([cat("gate", sl_self), cat("gate", sl_glob)], axis=0)
    Wg_n = cat("gate", sl_nbr)
    b1g = jnp.concatenate([heads[h]["gate"]["layers"][0][1] for h in range(HEADS)])
    W2g = jnp.zeros((HEADS * HID, HEADS), jnp.float32)
    for h in range(HEADS):
        W2g = W2g.at[h * HID:(h + 1) * HID, h].set(heads[h]["gate"]["out"][0][:, 0])
    b2g = jnp.stack([heads[h]["gate"]["out"][1][0] for h in range(HEADS)])
    Wm_s = jnp.concatenate([cat("msg", sl_self), cat("msg", sl_glob)], axis=0)
    Wm_n = cat("msg", sl_nbr)
    b1m = jnp.concatenate([heads[h]["msg"]["layers"][0][1] for h in range(HEADS)])
    W2m = jnp.stack([heads[h]["msg"]["out"][0] for h in range(HEADS)])
    b2m = jnp.stack([heads[h]["msg"]["out"][1].reshape(1, FEA) for h in range(HEADS)])
    return (Wg_s, Wg_n, b1g.reshape(1, -1), W2g, b2g.reshape(1, -1),
            Wm_s, Wm_n, b1m.reshape(1, -1), W2m, b2m)


def kernel(prec_weights, orig_prec_fea, self_fea_idx, nbr_fea_idx, reaction_prec_idx, actions, params):
    pw = prec_weights
    rxn = reaction_prec_idx
    fea, glob, tself = _embed(orig_prec_fea, pw, rxn, params["embed"], actions)

    sidx1 = self_fea_idx.astype(jnp.int32)
    nidx1 = nbr_fea_idx.astype(jnp.int32)
    pw1 = pw.reshape(N)
    zeros128 = jnp.zeros((N, FEA), jnp.float32)
    for heads in params["graphs"]:
        (Wg_s, Wg_n, b1g, W2g, b2g,
         Wm_s, Wm_n, b1m, W2m, b2m) = _pack_layer(heads)
        es, en = _sc_gather(tself, fea, sidx1, nidx1)
        gates = _gates(es, en, Wg_s, Wg_n, b1g, W2g, b2g)
        partials = _sc_segmax(gates, sidx1)
        gmax1 = _maxmerge(partials)
        w = _sc_weights(gates, gmax1, pw1, sidx1, nidx1)
        wmsg, wden = _msgs(es, en, w, Wm_s, Wm_n, b1m, W2m, b2m)
        onum, oden = _sc_acc(wmsg, wden, sidx1, zeros128)
        fea, tself = _norm(onum, oden, fea, glob)

    x = _cry(fea, pw, rxn, params["cry"])
    return _res_mlp(x, params["res"], params["out"])
